# Initial kernel scaffold; baseline (speedup 1.0000x reference)
import jax
import jax.numpy as jnp
from jax.experimental import pallas as pl


def kernel(node_feat, net_feat, pin_feat, pins_edge_index, pinned_edge_index, W_gc, b_gc, W_lin, b_lin, b_nn):
    raise NotImplementedError("write your pallas kernel here")


# speedup vs baseline: 2.1320x; 2.1320x over previous
import functools

import jax
import jax.numpy as jnp
from jax import lax
from jax.experimental import pallas as pl
from jax.experimental.pallas import tpu as pltpu
from jax.experimental.pallas import tpu_sc as plsc

N_NODE = 10000
N_NET = 10000
E = 160000
H_NODE = 128
H_NET = 16
H_PIN = 16
O_NODE = 16
O_NET = 128

NC = 2
NS = 16
NW = NC * NS
EPW = E // NW
CH = 128
NCH = EPW // CH
REM = EPW - NCH * CH

ROWS_A = 640
ROWS_B = N_NET - 15 * ROWS_A

def _worker_id():
    c = lax.axis_index("c")
    s = lax.axis_index("s")
    return c * NS + s, c, s


def _init_ones(ones_v):
    one = jnp.full((16,), 1.0, dtype=jnp.float32)
    for i in range(CH // 16):
        ones_v[pl.ds(i * 16, 16)] = one


def _fill_zero(ref):
    z = jnp.zeros((16,), dtype=jnp.float32)
    if len(ref.shape) == 1:
        for i in range(ref.shape[0] // 16):
            ref[pl.ds(i * 16, 16)] = z
    else:
        for r in range(ref.shape[0]):
            for j in range(ref.shape[1] // 16):
                ref[r, pl.ds(j * 16, 16)] = z


@functools.cache
def _build_sc_phase1():
  mesh = plsc.VectorSubcoreMesh(core_axis_name="c", subcore_axis_name="s")

  @functools.partial(
    pl.kernel,
    out_type=(
        jax.ShapeDtypeStruct((NC * 3 * N_NET,), jnp.float32),
        jax.ShapeDtypeStruct((E, H_NET), jnp.float32),
    ),
    mesh=mesh,
    compiler_params=pltpu.CompilerParams(use_tc_tiling_on_sc=False),
    scratch_types=(
        pltpu.VMEM((CH,), jnp.int32),
        pltpu.VMEM((REM,), jnp.int32),
        pltpu.VMEM((CH,), jnp.float32),
        pltpu.VMEM((CH, H_NET), jnp.float32),
        pltpu.VMEM((REM, H_NET), jnp.float32),
        pltpu.VMEM((80,), jnp.float32),
        pltpu.VMEM((80,), jnp.float32),
        pltpu.VMEM_SHARED((N_NODE,), jnp.float32),
        pltpu.VMEM_SHARED((N_NET,), jnp.float32),
        pltpu.VMEM_SHARED((N_NODE,), jnp.float32),
        pltpu.SemaphoreType.DMA,
    ),
  )
  def _sc_phase1(src_hbm, dst_hbm, s2_hbm, d2_hbm, net_feat_hbm,
                 deg_out, srch_out,
                 idx_v, idx8_v, ones_v, rows_v, rows8_v, zb_v, hb_v,
                 h0, h1, h2, sem):
    wid, c, s = _worker_id()
    _init_ones(ones_v)
    _fill_zero(zb_v)

    base = s * ROWS_A
    nt = jnp.where(s < 15, 8, 5)
    for h in (h0, h1, h2):
        def zbody(t, cr, _h=h):
            pltpu.sync_copy(zb_v, _h.at[pl.ds(base + t * 80, 80)])
            return cr
        lax.fori_loop(0, nt, zbody, 0)
    plsc.subcore_barrier()

    def chunk(base, idx, ones_src, rows):
        n = rows.shape[0]
        pltpu.sync_copy(src_hbm.at[pl.ds(base, n)], idx)
        pltpu.sync_copy(ones_src, h0.at[idx], add=True)
        pltpu.sync_copy(dst_hbm.at[pl.ds(base, n)], idx)
        pltpu.sync_copy(ones_src, h1.at[idx], add=True)
        pltpu.sync_copy(d2_hbm.at[pl.ds(base, n)], idx)
        pltpu.sync_copy(ones_src, h2.at[idx], add=True)
        pltpu.sync_copy(s2_hbm.at[pl.ds(base, n)], idx)
        pltpu.async_copy(net_feat_hbm.at[idx], rows, sem).wait()
        pltpu.sync_copy(rows, srch_out.at[pl.ds(base, n)])

    e0 = wid * EPW

    def body(j, carry):
        chunk(e0 + j * CH, idx_v, ones_v, rows_v)
        return carry

    lax.fori_loop(0, NCH, body, 0)
    chunk(e0 + NCH * CH, idx8_v, ones_v.at[pl.ds(0, REM)], rows8_v)

    plsc.subcore_barrier()
    for k, h in enumerate((h0, h1, h2)):
        def rbody(t, cr, _h=h, _k=k):
            o = base + t * 80
            pltpu.sync_copy(_h.at[pl.ds(o, 80)], hb_v)
            pltpu.sync_copy(hb_v, deg_out.at[
                pl.ds(pl.multiple_of(c * (3 * N_NET) + _k * N_NET + o, 8), 80)])
            return cr
        lax.fori_loop(0, nt, rbody, 0)

  return _sc_phase1


@functools.cache
def _build_sc_phase2():
  mesh = plsc.VectorSubcoreMesh(core_axis_name="c", subcore_axis_name="s")

  @functools.partial(
    pl.kernel,
    out_type=(
        jax.ShapeDtypeStruct((NC * N_NET, O_NET), jnp.float32),
        jax.ShapeDtypeStruct((NC * N_NODE, O_NODE), jnp.float32),
    ),
    mesh=mesh,
    compiler_params=pltpu.CompilerParams(use_tc_tiling_on_sc=False),
    scratch_types=(
        pltpu.VMEM((CH,), jnp.int32),
        pltpu.VMEM((CH,), jnp.int32),
        pltpu.VMEM((REM,), jnp.int32),
        pltpu.VMEM((REM,), jnp.int32),
        pltpu.VMEM((CH, O_NET), jnp.float32),
        pltpu.VMEM((REM, O_NET), jnp.float32),
        pltpu.VMEM((CH, O_NODE), jnp.float32),
        pltpu.VMEM((REM, O_NODE), jnp.float32),
        pltpu.VMEM((8, O_NET), jnp.float32),
        pltpu.VMEM((8, O_NODE), jnp.float32),
        pltpu.VMEM_SHARED((N_NET, O_NET), jnp.float32),
        pltpu.VMEM_SHARED((N_NODE, O_NODE), jnp.float32),
        pltpu.SemaphoreType.DMA,
    ),
  )
  def _sc_phase2(src_hbm, dst_hbm, d2_hbm, x_hbm, msg_hbm,
                 aggnet_out, aggnode_out,
                 gidx_v, sidx_v, gidx8_v, sidx8_v, rowsx_v, rowsx8_v,
                 rowsm_v, rowsm8_v, zrow_v, znrow_v, aggnet_s, aggnode_s, sem):
    wid, c, s = _worker_id()
    _fill_zero(zrow_v)
    _fill_zero(znrow_v)

    base = s * ROWS_A
    nz = jnp.where(s < 15, ROWS_A // 8, ROWS_B // 8)

    def zbody(t, cr):
        pltpu.sync_copy(zrow_v, aggnet_s.at[pl.ds(base + t * 8, 8)])
        pltpu.sync_copy(znrow_v, aggnode_s.at[pl.ds(base + t * 8, 8)])
        return cr

    lax.fori_loop(0, nz, zbody, 0)
    plsc.subcore_barrier()

    def chunk(base, gidx, sidx, rowsx, rowsm):
        n = rowsx.shape[0]
        pltpu.sync_copy(src_hbm.at[pl.ds(base, n)], gidx)
        pltpu.async_copy(x_hbm.at[gidx], rowsx, sem).wait()
        pltpu.sync_copy(dst_hbm.at[pl.ds(base, n)], sidx)
        pltpu.sync_copy(rowsx, aggnet_s.at[sidx], add=True)
        pltpu.sync_copy(msg_hbm.at[pl.ds(base, n)], rowsm)
        pltpu.sync_copy(d2_hbm.at[pl.ds(base, n)], gidx)
        pltpu.sync_copy(rowsm, aggnode_s.at[gidx], add=True)

    e0 = wid * EPW

    def body(j, carry):
        chunk(e0 + j * CH, gidx_v, sidx_v, rowsx_v, rowsm_v)
        return carry

    lax.fori_loop(0, NCH, body, 0)
    chunk(e0 + NCH * CH, gidx8_v, sidx8_v, rowsx8_v, rowsm8_v)

    plsc.subcore_barrier()
    nr = jnp.where(s < 15, 5, 3)

    def rbody(t, cr):
        o = base + t * CH
        oo = pl.multiple_of(c * N_NET + o, 8)
        pltpu.sync_copy(aggnet_s.at[pl.ds(o, CH)], rowsx_v)
        pltpu.sync_copy(rowsx_v, aggnet_out.at[pl.ds(oo, CH)])
        pltpu.sync_copy(aggnode_s.at[pl.ds(o, CH)], rowsm_v)
        pltpu.sync_copy(rowsm_v, aggnode_out.at[pl.ds(oo, CH)])
        return cr

    lax.fori_loop(0, nr, rbody, 0)

    @pl.when(s == 15)
    def _():
        o = 15 * ROWS_A + 3 * CH
        oo = pl.multiple_of(c * N_NET + o, 8)
        pltpu.sync_copy(aggnet_s.at[pl.ds(o, 16)], rowsx_v.at[pl.ds(0, 16)])
        pltpu.sync_copy(rowsx_v.at[pl.ds(0, 16)], aggnet_out.at[pl.ds(oo, 16)])
        pltpu.sync_copy(aggnode_s.at[pl.ds(o, 16)], rowsm_v.at[pl.ds(0, 16)])
        pltpu.sync_copy(rowsm_v.at[pl.ds(0, 16)], aggnode_out.at[pl.ds(oo, 16)])

  return _sc_phase2


_NB = 2000


def _tc_x_body(od_ref, nf_ref, x_ref):
    d = jnp.sum(od_ref[...], axis=1, keepdims=True)
    norm = jnp.where(d > 0.0, lax.rsqrt(jnp.maximum(d, 1.0)), 0.0)
    x_ref[...] = nf_ref[...] * norm


def _tc_msg_body(pin_ref, srch_ref, w2_ref, b2_ref, msg_ref):
    pin = pin_ref[...]
    srch = srch_ref[...]
    z = jnp.concatenate([pin[:, p:p + 1] * srch for p in range(H_PIN)], axis=1)
    msg_ref[...] = (
        jnp.dot(z, w2_ref[...], preferred_element_type=jnp.float32)
        + jnp.dot(srch, b2_ref[...], preferred_element_type=jnp.float32)
    )


def _tc_final_body(anet_ref, ind_ref, anode_ref, d2_ref, wgc_ref, bgc_ref,
                   bnn_ref, hnet_ref, hnode_ref):
    ind = jnp.sum(ind_ref[...], axis=1, keepdims=True)
    norm = jnp.where(ind > 0.0, lax.rsqrt(jnp.maximum(ind, 1.0)), 0.0)
    anet = (anet_ref[0] + anet_ref[1]) * norm
    hnet_ref[...] = (
        jnp.dot(anet, wgc_ref[...], preferred_element_type=jnp.float32)
        + bgc_ref[...]
    )
    dg = jnp.maximum(jnp.sum(d2_ref[...], axis=1, keepdims=True), 1.0)
    hnode_ref[...] = (anode_ref[0] + anode_ref[1]) / dg + bnn_ref[...]


def kernel(node_feat, net_feat, pin_feat, pins_edge_index, pinned_edge_index,
           W_gc, b_gc, W_lin, b_lin, b_nn):
    idx1 = pins_edge_index.astype(jnp.int32)
    idx2 = pinned_edge_index.astype(jnp.int32)
    src, dst = idx1[0], idx1[1]
    s2, d2 = idx2[0], idx2[1]

    deg_flat, src_h = _build_sc_phase1()(src, dst, s2, d2, net_feat)
    deg_p = deg_flat.reshape(NC, 3, N_NET)

    x = pl.pallas_call(
        _tc_x_body,
        grid=(N_NODE // _NB,),
        in_specs=[
            pl.BlockSpec((_NB, NC), lambda i: (i, 0)),
            pl.BlockSpec((_NB, H_NODE), lambda i: (i, 0)),
        ],
        out_specs=pl.BlockSpec((_NB, H_NODE), lambda i: (i, 0)),
        out_shape=jax.ShapeDtypeStruct((N_NODE, H_NODE), jnp.float32),
    )(deg_p[:, 0, :].T, node_feat)

    w2 = W_lin.reshape(H_PIN * H_NET, O_NODE)
    b2 = b_lin.reshape(H_NET, O_NODE)
    msg = pl.pallas_call(
        _tc_msg_body,
        grid=(E // _NB,),
        in_specs=[
            pl.BlockSpec((_NB, H_PIN), lambda i: (i, 0)),
            pl.BlockSpec((_NB, H_NET), lambda i: (i, 0)),
            pl.BlockSpec((H_PIN * H_NET, O_NODE), lambda i: (0, 0)),
            pl.BlockSpec((H_NET, O_NODE), lambda i: (0, 0)),
        ],
        out_specs=pl.BlockSpec((_NB, O_NODE), lambda i: (i, 0)),
        out_shape=jax.ShapeDtypeStruct((E, O_NODE), jnp.float32),
    )(pin_feat, src_h, w2, b2)

    aggnet_f, aggnode_f = _build_sc_phase2()(src, dst, d2, x, msg)
    aggnet_p = aggnet_f.reshape(NC, N_NET, O_NET)
    aggnode_p = aggnode_f.reshape(NC, N_NODE, O_NODE)

    h_net, h_node = pl.pallas_call(
        _tc_final_body,
        grid=(N_NET // _NB,),
        in_specs=[
            pl.BlockSpec((NC, _NB, O_NET), lambda i: (0, i, 0)),
            pl.BlockSpec((_NB, NC), lambda i: (i, 0)),
            pl.BlockSpec((NC, _NB, O_NODE), lambda i: (0, i, 0)),
            pl.BlockSpec((_NB, NC), lambda i: (i, 0)),
            pl.BlockSpec((H_NODE, O_NET), lambda i: (0, 0)),
            pl.BlockSpec((1, O_NET), lambda i: (0, 0)),
            pl.BlockSpec((1, O_NODE), lambda i: (0, 0)),
        ],
        out_specs=[
            pl.BlockSpec((_NB, O_NET), lambda i: (i, 0)),
            pl.BlockSpec((_NB, O_NODE), lambda i: (i, 0)),
        ],
        out_shape=[
            jax.ShapeDtypeStruct((N_NET, O_NET), jnp.float32),
            jax.ShapeDtypeStruct((N_NODE, O_NODE), jnp.float32),
        ],
    )(aggnet_p, deg_p[:, 1, :].T, aggnode_p, deg_p[:, 2, :].T,
      W_gc, b_gc.reshape(1, O_NET), b_nn.reshape(1, O_NODE))

    return (h_node, h_net)


# speedup vs baseline: 3.7420x; 1.7552x over previous
import functools

import jax
import jax.numpy as jnp
from jax import lax
from jax.experimental import pallas as pl
from jax.experimental.pallas import tpu as pltpu
from jax.experimental.pallas import tpu_sc as plsc

N_NODE = 10000
N_NET = 10000
E = 160000
H_NODE = 128
H_NET = 16
H_PIN = 16
O_NODE = 16
O_NET = 128

NC = 2
NS = 16
NW = NC * NS
EPW = E // NW
CH = 128
NCH = EPW // CH
REM = EPW - NCH * CH

ROWS_A = 640
ROWS_B = N_NET - 15 * ROWS_A

def _worker_id():
    c = lax.axis_index("c")
    s = lax.axis_index("s")
    return c * NS + s, c, s


def _init_ones(ones_v):
    one = jnp.full((16,), 1.0, dtype=jnp.float32)
    for i in range(CH // 16):
        ones_v[pl.ds(i * 16, 16)] = one


def _fill_zero(ref):
    z = jnp.zeros((16,), dtype=jnp.float32)
    if len(ref.shape) == 1:
        for i in range(ref.shape[0] // 16):
            ref[pl.ds(i * 16, 16)] = z
    else:
        for r in range(ref.shape[0]):
            for j in range(ref.shape[1] // 16):
                ref[r, pl.ds(j * 16, 16)] = z


@functools.cache
def _build_sc_phase1():
  mesh = plsc.VectorSubcoreMesh(core_axis_name="c", subcore_axis_name="s")

  @functools.partial(
    pl.kernel,
    out_type=(
        jax.ShapeDtypeStruct((NC * 3 * N_NET,), jnp.float32),
        jax.ShapeDtypeStruct((E, H_NET), jnp.float32),
    ),
    mesh=mesh,
    compiler_params=pltpu.CompilerParams(use_tc_tiling_on_sc=False),
    scratch_types=(
        pltpu.VMEM((CH,), jnp.int32),
        pltpu.VMEM((REM,), jnp.int32),
        pltpu.VMEM((CH,), jnp.float32),
        pltpu.VMEM((CH, H_NET), jnp.float32),
        pltpu.VMEM((REM, H_NET), jnp.float32),
        pltpu.VMEM((80,), jnp.float32),
        pltpu.VMEM((80,), jnp.float32),
        pltpu.VMEM_SHARED((N_NODE,), jnp.float32),
        pltpu.VMEM_SHARED((N_NET,), jnp.float32),
        pltpu.VMEM_SHARED((N_NODE,), jnp.float32),
        pltpu.SemaphoreType.DMA,
    ),
  )
  def _sc_phase1(src_hbm, dst_hbm, s2_hbm, d2_hbm, net_feat_hbm,
                 deg_out, srch_out,
                 idx_v, idx8_v, ones_v, rows_v, rows8_v, zb_v, hb_v,
                 h0, h1, h2, sem):
    wid, c, s = _worker_id()
    _init_ones(ones_v)
    _fill_zero(zb_v)

    base = s * ROWS_A
    nt = jnp.where(s < 15, 8, 5)
    for h in (h0, h1, h2):
        def zbody(t, cr, _h=h):
            pltpu.sync_copy(zb_v, _h.at[pl.ds(base + t * 80, 80)])
            return cr
        lax.fori_loop(0, nt, zbody, 0)
    plsc.subcore_barrier()

    def chunk(base, idx, ones_src, rows):
        n = rows.shape[0]
        pltpu.sync_copy(src_hbm.at[pl.ds(base, n)], idx)
        pltpu.sync_copy(ones_src, h0.at[idx], add=True)
        pltpu.sync_copy(dst_hbm.at[pl.ds(base, n)], idx)
        pltpu.sync_copy(ones_src, h1.at[idx], add=True)
        pltpu.sync_copy(d2_hbm.at[pl.ds(base, n)], idx)
        pltpu.sync_copy(ones_src, h2.at[idx], add=True)
        pltpu.sync_copy(s2_hbm.at[pl.ds(base, n)], idx)
        pltpu.async_copy(net_feat_hbm.at[idx], rows, sem).wait()
        pltpu.sync_copy(rows, srch_out.at[pl.ds(base, n)])

    e0 = wid * EPW

    def body(j, carry):
        chunk(e0 + j * CH, idx_v, ones_v, rows_v)
        return carry

    lax.fori_loop(0, NCH, body, 0)
    chunk(e0 + NCH * CH, idx8_v, ones_v.at[pl.ds(0, REM)], rows8_v)

    plsc.subcore_barrier()
    for k, h in enumerate((h0, h1, h2)):
        def rbody(t, cr, _h=h, _k=k):
            o = base + t * 80
            pltpu.sync_copy(_h.at[pl.ds(o, 80)], hb_v)
            pltpu.sync_copy(hb_v, deg_out.at[
                pl.ds(pl.multiple_of(c * (3 * N_NET) + _k * N_NET + o, 8), 80)])
            return cr
        lax.fori_loop(0, nt, rbody, 0)

  return _sc_phase1


@functools.cache
def _build_sc_phase2():
  mesh = plsc.VectorSubcoreMesh(core_axis_name="c", subcore_axis_name="s")

  @functools.partial(
    pl.kernel,
    out_type=(
        jax.ShapeDtypeStruct((NC * N_NET, O_NET), jnp.float32),
        jax.ShapeDtypeStruct((NC * N_NODE, O_NODE), jnp.float32),
    ),
    mesh=mesh,
    compiler_params=pltpu.CompilerParams(use_tc_tiling_on_sc=False),
    scratch_types=(
        pltpu.VMEM((CH,), jnp.int32),
        pltpu.VMEM((CH,), jnp.int32),
        pltpu.VMEM((REM,), jnp.int32),
        pltpu.VMEM((REM,), jnp.int32),
        pltpu.VMEM((CH, O_NET), jnp.float32),
        pltpu.VMEM((REM, O_NET), jnp.float32),
        pltpu.VMEM((CH, O_NODE), jnp.float32),
        pltpu.VMEM((REM, O_NODE), jnp.float32),
        pltpu.VMEM((8, O_NET), jnp.float32),
        pltpu.VMEM((8, O_NODE), jnp.float32),
        pltpu.VMEM_SHARED((N_NET, O_NET), jnp.float32),
        pltpu.VMEM_SHARED((N_NODE, O_NODE), jnp.float32),
        pltpu.SemaphoreType.DMA,
    ),
  )
  def _sc_phase2(src_hbm, dst_hbm, d2_hbm, x_hbm, msg_hbm,
                 aggnet_out, aggnode_out,
                 gidx_v, sidx_v, gidx8_v, sidx8_v, rowsx_v, rowsx8_v,
                 rowsm_v, rowsm8_v, zrow_v, znrow_v, aggnet_s, aggnode_s, sem):
    wid, c, s = _worker_id()
    _fill_zero(zrow_v)
    _fill_zero(znrow_v)

    base = s * ROWS_A
    nz = jnp.where(s < 15, ROWS_A // 8, ROWS_B // 8)

    def zbody(t, cr):
        pltpu.sync_copy(zrow_v, aggnet_s.at[pl.ds(base + t * 8, 8)])
        pltpu.sync_copy(znrow_v, aggnode_s.at[pl.ds(base + t * 8, 8)])
        return cr

    lax.fori_loop(0, nz, zbody, 0)
    plsc.subcore_barrier()

    def chunk(base, gidx, sidx, rowsx, rowsm):
        n = rowsx.shape[0]
        pltpu.sync_copy(src_hbm.at[pl.ds(base, n)], gidx)
        pltpu.async_copy(x_hbm.at[gidx], rowsx, sem).wait()
        pltpu.sync_copy(dst_hbm.at[pl.ds(base, n)], sidx)
        pltpu.sync_copy(rowsx, aggnet_s.at[sidx], add=True)
        pltpu.sync_copy(msg_hbm.at[pl.ds(base, n)], rowsm)
        pltpu.sync_copy(d2_hbm.at[pl.ds(base, n)], gidx)
        pltpu.sync_copy(rowsm, aggnode_s.at[gidx], add=True)

    e0 = wid * EPW

    def body(j, carry):
        chunk(e0 + j * CH, gidx_v, sidx_v, rowsx_v, rowsm_v)
        return carry

    lax.fori_loop(0, NCH, body, 0)
    chunk(e0 + NCH * CH, gidx8_v, sidx8_v, rowsx8_v, rowsm8_v)

    plsc.subcore_barrier()
    nr = jnp.where(s < 15, 5, 3)

    def rbody(t, cr):
        o = base + t * CH
        oo = pl.multiple_of(c * N_NET + o, 8)
        pltpu.sync_copy(aggnet_s.at[pl.ds(o, CH)], rowsx_v)
        pltpu.sync_copy(rowsx_v, aggnet_out.at[pl.ds(oo, CH)])
        pltpu.sync_copy(aggnode_s.at[pl.ds(o, CH)], rowsm_v)
        pltpu.sync_copy(rowsm_v, aggnode_out.at[pl.ds(oo, CH)])
        return cr

    lax.fori_loop(0, nr, rbody, 0)

    @pl.when(s == 15)
    def _():
        o = 15 * ROWS_A + 3 * CH
        oo = pl.multiple_of(c * N_NET + o, 8)
        pltpu.sync_copy(aggnet_s.at[pl.ds(o, 16)], rowsx_v.at[pl.ds(0, 16)])
        pltpu.sync_copy(rowsx_v.at[pl.ds(0, 16)], aggnet_out.at[pl.ds(oo, 16)])
        pltpu.sync_copy(aggnode_s.at[pl.ds(o, 16)], rowsm_v.at[pl.ds(0, 16)])
        pltpu.sync_copy(rowsm_v.at[pl.ds(0, 16)], aggnode_out.at[pl.ds(oo, 16)])

  return _sc_phase2


_NB = 2000


def _tc_x_body(od_ref, nf_ref, x_ref):
    d = jnp.sum(od_ref[...], axis=1, keepdims=True)
    norm = jnp.where(d > 0.0, lax.rsqrt(jnp.maximum(d, 1.0)), 0.0)
    x_ref[...] = nf_ref[...] * norm


def _tc_msg_body(pin_ref, srch_ref, a_ref, c_ref, w2_ref, b2_ref, msg_ref):
    pin = pin_ref[...]
    srch = srch_ref[...]
    pr = jnp.dot(pin, a_ref[...], preferred_element_type=jnp.float32)
    st = jnp.dot(srch, c_ref[...], preferred_element_type=jnp.float32)
    msg_ref[...] = (
        jnp.dot(pr * st, w2_ref[...], preferred_element_type=jnp.float32)
        + jnp.dot(srch, b2_ref[...], preferred_element_type=jnp.float32)
    )


def _tc_final_body(anet_ref, ind_ref, anode_ref, d2_ref, wgc_ref, bgc_ref,
                   bnn_ref, hnet_ref, hnode_ref):
    ind = jnp.sum(ind_ref[...], axis=1, keepdims=True)
    norm = jnp.where(ind > 0.0, lax.rsqrt(jnp.maximum(ind, 1.0)), 0.0)
    anet = (anet_ref[0] + anet_ref[1]) * norm
    hnet_ref[...] = (
        jnp.dot(anet, wgc_ref[...], preferred_element_type=jnp.float32)
        + bgc_ref[...]
    )
    dg = jnp.maximum(jnp.sum(d2_ref[...], axis=1, keepdims=True), 1.0)
    hnode_ref[...] = (anode_ref[0] + anode_ref[1]) / dg + bnn_ref[...]


def kernel(node_feat, net_feat, pin_feat, pins_edge_index, pinned_edge_index,
           W_gc, b_gc, W_lin, b_lin, b_nn):
    idx1 = pins_edge_index.astype(jnp.int32)
    idx2 = pinned_edge_index.astype(jnp.int32)
    src, dst = idx1[0], idx1[1]
    s2, d2 = idx2[0], idx2[1]

    deg_flat, src_h = _build_sc_phase1()(src, dst, s2, d2, net_feat)
    deg_p = deg_flat.reshape(NC, 3, N_NET)

    x = pl.pallas_call(
        _tc_x_body,
        grid=(N_NODE // _NB,),
        in_specs=[
            pl.BlockSpec((_NB, NC), lambda i: (i, 0)),
            pl.BlockSpec((_NB, H_NODE), lambda i: (i, 0)),
        ],
        out_specs=pl.BlockSpec((_NB, H_NODE), lambda i: (i, 0)),
        out_shape=jax.ShapeDtypeStruct((N_NODE, H_NODE), jnp.float32),
    )(deg_p[:, 0, :].T, node_feat)

    w2 = W_lin.reshape(H_PIN * H_NET, O_NODE)
    b2 = b_lin.reshape(H_NET, O_NODE)
    eye = jnp.eye(H_PIN, dtype=jnp.float32)
    a_sel = jnp.repeat(eye, H_NET, axis=1)
    c_sel = jnp.tile(eye, (1, H_PIN))
    msg = pl.pallas_call(
        _tc_msg_body,
        grid=(E // _NB,),
        in_specs=[
            pl.BlockSpec((_NB, H_PIN), lambda i: (i, 0)),
            pl.BlockSpec((_NB, H_NET), lambda i: (i, 0)),
            pl.BlockSpec((H_PIN, H_PIN * H_NET), lambda i: (0, 0)),
            pl.BlockSpec((H_NET, H_PIN * H_NET), lambda i: (0, 0)),
            pl.BlockSpec((H_PIN * H_NET, O_NODE), lambda i: (0, 0)),
            pl.BlockSpec((H_NET, O_NODE), lambda i: (0, 0)),
        ],
        out_specs=pl.BlockSpec((_NB, O_NODE), lambda i: (i, 0)),
        out_shape=jax.ShapeDtypeStruct((E, O_NODE), jnp.float32),
    )(pin_feat, src_h, a_sel, c_sel, w2, b2)

    aggnet_f, aggnode_f = _build_sc_phase2()(src, dst, d2, x, msg)
    aggnet_p = aggnet_f.reshape(NC, N_NET, O_NET)
    aggnode_p = aggnode_f.reshape(NC, N_NODE, O_NODE)

    h_net, h_node = pl.pallas_call(
        _tc_final_body,
        grid=(N_NET // _NB,),
        in_specs=[
            pl.BlockSpec((NC, _NB, O_NET), lambda i: (0, i, 0)),
            pl.BlockSpec((_NB, NC), lambda i: (i, 0)),
            pl.BlockSpec((NC, _NB, O_NODE), lambda i: (0, i, 0)),
            pl.BlockSpec((_NB, NC), lambda i: (i, 0)),
            pl.BlockSpec((H_NODE, O_NET), lambda i: (0, 0)),
            pl.BlockSpec((1, O_NET), lambda i: (0, 0)),
            pl.BlockSpec((1, O_NODE), lambda i: (0, 0)),
        ],
        out_specs=[
            pl.BlockSpec((_NB, O_NET), lambda i: (i, 0)),
            pl.BlockSpec((_NB, O_NODE), lambda i: (i, 0)),
        ],
        out_shape=[
            jax.ShapeDtypeStruct((N_NET, O_NET), jnp.float32),
            jax.ShapeDtypeStruct((N_NODE, O_NODE), jnp.float32),
        ],
    )(aggnet_p, deg_p[:, 1, :].T, aggnode_p, deg_p[:, 2, :].T,
      W_gc, b_gc.reshape(1, O_NET), b_nn.reshape(1, O_NODE))

    return (h_node, h_net)


# speedup vs baseline: 5.0333x; 1.3451x over previous
import functools

import jax
import jax.numpy as jnp
from jax import lax
from jax.experimental import pallas as pl
from jax.experimental.pallas import tpu as pltpu
from jax.experimental.pallas import tpu_sc as plsc

N_NODE = 10000
N_NET = 10000
E = 160000
H_NODE = 128
H_NET = 16
H_PIN = 16
O_NODE = 16
O_NET = 128

NC = 2
NS = 16
NW = NC * NS
EPW = E // NW
CH = 128
NCH = EPW // CH
REM = EPW - NCH * CH
NB = 3
CH2 = 96
NCH2 = EPW // CH2
REM2 = EPW - NCH2 * CH2
RD = 80

ROWS_A = 640
ROWS_B = N_NET - 15 * ROWS_A


def _worker_id():
    c = lax.axis_index("c")
    s = lax.axis_index("s")
    return c * NS + s, c, s


def _init_ones(ones_v):
    one = jnp.full((16,), 1.0, dtype=jnp.float32)
    for i in range(ones_v.shape[0] // 16):
        ones_v[pl.ds(i * 16, 16)] = one


def _fill_zero(ref):
    z = jnp.zeros((16,), dtype=jnp.float32)
    if len(ref.shape) == 1:
        for i in range(ref.shape[0] // 16):
            ref[pl.ds(i * 16, 16)] = z
    else:
        for r in range(ref.shape[0]):
            for j in range(ref.shape[1] // 16):
                ref[r, pl.ds(j * 16, 16)] = z


def _zero_hist(h, zb_v, s):
    base = s * ROWS_A
    nt = jnp.where(s < 15, 8, 5)

    def zbody(t, cr):
        pltpu.sync_copy(zb_v, h.at[pl.ds(base + t * 80, 80)])
        return cr

    lax.fori_loop(0, nt, zbody, 0)


def _read_hist(h, hb_v, out, out_base, s):
    base = s * ROWS_A
    nt = jnp.where(s < 15, 8, 5)

    def rbody(t, cr):
        o = base + t * 80
        pltpu.sync_copy(h.at[pl.ds(o, 80)], hb_v)
        pltpu.sync_copy(hb_v, out.at[pl.ds(pl.multiple_of(out_base + o, 8), 80)])
        return cr

    lax.fori_loop(0, nt, rbody, 0)


@functools.cache
def _build_sc_phase1():
  mesh = plsc.VectorSubcoreMesh(core_axis_name="c", subcore_axis_name="s")

  @functools.partial(
    pl.kernel,
    out_type=(
        jax.ShapeDtypeStruct((NC * N_NODE,), jnp.float32),
        jax.ShapeDtypeStruct((E, H_NET), jnp.float32),
    ),
    mesh=mesh,
    compiler_params=pltpu.CompilerParams(use_tc_tiling_on_sc=False),
    scratch_types=(
        pltpu.VMEM((NB, CH), jnp.int32),
        pltpu.VMEM((NB, CH), jnp.int32),
        pltpu.VMEM((NB, CH, H_NET), jnp.float32),
        pltpu.VMEM((REM,), jnp.int32),
        pltpu.VMEM((REM,), jnp.int32),
        pltpu.VMEM((REM, H_NET), jnp.float32),
        pltpu.VMEM((CH,), jnp.float32),
        pltpu.VMEM((80,), jnp.float32),
        pltpu.VMEM((80,), jnp.float32),
        pltpu.VMEM_SHARED((N_NODE,), jnp.float32),
        pltpu.SemaphoreType.DMA((NB,)),
        pltpu.SemaphoreType.DMA((NB,)),
        pltpu.SemaphoreType.DMA((NB,)),
        pltpu.SemaphoreType.DMA((NB,)),
        pltpu.SemaphoreType.DMA((NB,)),
        pltpu.SemaphoreType.DMA,
    ),
  )
  def _sc_phase1(src_hbm, s2_hbm, net_feat_hbm,
                 od_out, srch_out,
                 isrc, is2, rows, r_isrc, r_is2, r_rows, ones_v, zb_v, hb_v,
                 h0, sem_ls, sem_l2, sem_h, sem_g, sem_w, sem_r):
    wid, c, s = _worker_id()
    _init_ones(ones_v)
    _fill_zero(zb_v)
    _zero_hist(h0, zb_v, s)
    plsc.subcore_barrier()

    e0 = wid * EPW
    d_ls = [None] * NCH
    d_l2 = [None] * NCH
    d_h = [None] * NCH
    d_g = [None] * NCH
    d_w = [None] * NCH

    for t in range(NCH + 2):
        if t < NCH:
            b = t % NB
            if t >= NB:
                d_h[t - NB].wait()
                d_w[t - NB].wait()
            base = e0 + t * CH
            d_ls[t] = pltpu.async_copy(
                src_hbm.at[pl.ds(base, CH)], isrc.at[b], sem_ls.at[b])
            d_l2[t] = pltpu.async_copy(
                s2_hbm.at[pl.ds(base, CH)], is2.at[b], sem_l2.at[b])
        j = t - 1
        if 0 <= j < NCH:
            b = j % NB
            d_ls[j].wait()
            d_h[j] = pltpu.async_copy(ones_v, h0.at[isrc.at[b]], sem_h.at[b],
                                      add=True)
            d_l2[j].wait()
            d_g[j] = pltpu.async_copy(net_feat_hbm.at[is2.at[b]], rows.at[b],
                                      sem_g.at[b])
        j = t - 2
        if j >= 0:
            b = j % NB
            d_g[j].wait()
            d_w[j] = pltpu.async_copy(rows.at[b],
                                      srch_out.at[pl.ds(e0 + j * CH, CH)],
                                      sem_w.at[b])

    for j in range(NCH - NB, NCH):
        d_h[j].wait()
        d_w[j].wait()

    rbase = e0 + NCH * CH
    pltpu.sync_copy(src_hbm.at[pl.ds(rbase, REM)], r_isrc)
    pltpu.sync_copy(ones_v.at[pl.ds(0, REM)], h0.at[r_isrc], add=True)
    pltpu.sync_copy(s2_hbm.at[pl.ds(rbase, REM)], r_is2)
    pltpu.async_copy(net_feat_hbm.at[r_is2], r_rows, sem_r).wait()
    pltpu.sync_copy(r_rows, srch_out.at[pl.ds(rbase, REM)])

    plsc.subcore_barrier()
    _read_hist(h0, hb_v, od_out, c * N_NODE, s)

  return _sc_phase1


@functools.cache
def _build_sc_phase2():
  mesh = plsc.VectorSubcoreMesh(core_axis_name="c", subcore_axis_name="s")

  @functools.partial(
    pl.kernel,
    out_type=(
        jax.ShapeDtypeStruct((NC * N_NET, O_NET), jnp.float32),
        jax.ShapeDtypeStruct((NC * N_NODE, O_NODE), jnp.float32),
        jax.ShapeDtypeStruct((NC * 2 * N_NET,), jnp.float32),
    ),
    mesh=mesh,
    compiler_params=pltpu.CompilerParams(use_tc_tiling_on_sc=False),
    scratch_types=(
        pltpu.VMEM((NB, CH2), jnp.int32),
        pltpu.VMEM((NB, CH2), jnp.int32),
        pltpu.VMEM((NB, CH2), jnp.int32),
        pltpu.VMEM((2, CH2, O_NET), jnp.float32),
        pltpu.VMEM((2, CH2, O_NODE), jnp.float32),
        pltpu.VMEM((REM2,), jnp.int32),
        pltpu.VMEM((REM2,), jnp.int32),
        pltpu.VMEM((REM2,), jnp.int32),
        pltpu.VMEM((REM2, O_NET), jnp.float32),
        pltpu.VMEM((REM2, O_NODE), jnp.float32),
        pltpu.VMEM((CH2,), jnp.float32),
        pltpu.VMEM((8, O_NET), jnp.float32),
        pltpu.VMEM((8, O_NODE), jnp.float32),
        pltpu.VMEM((80,), jnp.float32),
        pltpu.VMEM((80,), jnp.float32),
        pltpu.VMEM_SHARED((N_NET, O_NET), jnp.float32),
        pltpu.VMEM_SHARED((N_NODE, O_NODE), jnp.float32),
        pltpu.VMEM_SHARED((N_NET,), jnp.float32),
        pltpu.VMEM_SHARED((N_NODE,), jnp.float32),
        pltpu.SemaphoreType.DMA((NB,)),
        pltpu.SemaphoreType.DMA((NB,)),
        pltpu.SemaphoreType.DMA((NB,)),
        pltpu.SemaphoreType.DMA((NB,)),
        pltpu.SemaphoreType.DMA((NB,)),
        pltpu.SemaphoreType.DMA((NB,)),
        pltpu.SemaphoreType.DMA((NB,)),
        pltpu.SemaphoreType.DMA((NB,)),
        pltpu.SemaphoreType.DMA((NB,)),
        pltpu.SemaphoreType.DMA,
    ),
  )
  def _sc_phase2(src_hbm, dst_hbm, d2_hbm, x_hbm, msg_hbm,
                 aggnet_out, aggnode_out, deg_out,
                 isrc, idst, id2, rx, rm, r_isrc, r_idst, r_id2, r_rx, r_rm,
                 ones_v, zrow_v, znrow_v, zb_v, hb_v,
                 aggnet_s, aggnode_s, h1, h2,
                 sem_ls, sem_ld, sem_l2, sem_lm, sem_g, sem_sn, sem_sd,
                 sem_h1, sem_h2, sem_r):
    wid, c, s = _worker_id()
    _init_ones(ones_v)
    _fill_zero(zrow_v)
    _fill_zero(znrow_v)
    _fill_zero(zb_v)
    _zero_hist(h1, zb_v, s)
    _zero_hist(h2, zb_v, s)

    base = s * ROWS_A
    nz = jnp.where(s < 15, ROWS_A // 8, ROWS_B // 8)

    def zbody(t, cr):
        pltpu.sync_copy(zrow_v, aggnet_s.at[pl.ds(base + t * 8, 8)])
        pltpu.sync_copy(znrow_v, aggnode_s.at[pl.ds(base + t * 8, 8)])
        return cr

    lax.fori_loop(0, nz, zbody, 0)
    plsc.subcore_barrier()

    e0 = wid * EPW
    d_ls = [None] * NCH2
    d_ld = [None] * NCH2
    d_l2 = [None] * NCH2
    d_lm = [None] * NCH2
    d_g = [None] * NCH2
    d_sn = [None] * NCH2
    d_sd = [None] * NCH2
    d_h1 = [None] * NCH2
    d_h2 = [None] * NCH2

    for t in range(NCH2 + 2):
        j = t - 2
        if j >= 0:
            b = j % NB
            d_g[j].wait()
            d_sn[j] = pltpu.async_copy(rx.at[j % 2], aggnet_s.at[idst.at[b]],
                                       sem_sn.at[b], add=True)
            d_lm[j].wait()
            d_sd[j] = pltpu.async_copy(rm.at[j % 2], aggnode_s.at[id2.at[b]],
                                       sem_sd.at[b], add=True)
        if t < NCH2:
            b = t % NB
            if t >= 2:
                d_sd[t - 2].wait()
            if t >= NB:
                j0 = t - NB
                d_sn[j0].wait()
                d_h1[j0].wait()
                d_h2[j0].wait()
            eb = e0 + t * CH2
            d_ls[t] = pltpu.async_copy(
                src_hbm.at[pl.ds(eb, CH2)], isrc.at[b], sem_ls.at[b])
            d_ld[t] = pltpu.async_copy(
                dst_hbm.at[pl.ds(eb, CH2)], idst.at[b], sem_ld.at[b])
            d_l2[t] = pltpu.async_copy(
                d2_hbm.at[pl.ds(eb, CH2)], id2.at[b], sem_l2.at[b])
            d_lm[t] = pltpu.async_copy(
                msg_hbm.at[pl.ds(eb, CH2)], rm.at[t % 2], sem_lm.at[t % 2])
        j = t - 1
        if 0 <= j < NCH2:
            b = j % NB
            d_ls[j].wait()
            d_g[j] = pltpu.async_copy(x_hbm.at[isrc.at[b]], rx.at[j % 2],
                                      sem_g.at[j % 2])
            d_ld[j].wait()
            d_h1[j] = pltpu.async_copy(ones_v, h1.at[idst.at[b]],
                                       sem_h1.at[b], add=True)
            d_l2[j].wait()
            d_h2[j] = pltpu.async_copy(ones_v, h2.at[id2.at[b]],
                                       sem_h2.at[b], add=True)

    for j in range(NCH2 - NB, NCH2):
        d_sn[j].wait()
        d_h1[j].wait()
        d_h2[j].wait()
    for j in range(NCH2 - 2, NCH2):
        d_sd[j].wait()

    rbase = e0 + NCH2 * CH2
    pltpu.sync_copy(src_hbm.at[pl.ds(rbase, REM2)], r_isrc)
    pltpu.sync_copy(dst_hbm.at[pl.ds(rbase, REM2)], r_idst)
    pltpu.sync_copy(d2_hbm.at[pl.ds(rbase, REM2)], r_id2)
    pltpu.async_copy(x_hbm.at[r_isrc], r_rx, sem_r).wait()
    pltpu.sync_copy(r_rx, aggnet_s.at[r_idst], add=True)
    pltpu.sync_copy(ones_v.at[pl.ds(0, REM2)], h1.at[r_idst], add=True)
    pltpu.sync_copy(msg_hbm.at[pl.ds(rbase, REM2)], r_rm)
    pltpu.sync_copy(r_rm, aggnode_s.at[r_id2], add=True)
    pltpu.sync_copy(ones_v.at[pl.ds(0, REM2)], h2.at[r_id2], add=True)

    plsc.subcore_barrier()
    nr = jnp.where(s < 15, ROWS_A // RD, ROWS_B // RD)

    def rbody(t, cr):
        o = base + t * RD
        oo = pl.multiple_of(c * N_NET + o, 8)
        pltpu.sync_copy(aggnet_s.at[pl.ds(o, RD)], rx.at[0].at[pl.ds(0, RD)])
        pltpu.sync_copy(rx.at[0].at[pl.ds(0, RD)],
                        aggnet_out.at[pl.ds(oo, RD)])
        pltpu.sync_copy(aggnode_s.at[pl.ds(o, RD)], rm.at[0].at[pl.ds(0, RD)])
        pltpu.sync_copy(rm.at[0].at[pl.ds(0, RD)],
                        aggnode_out.at[pl.ds(oo, RD)])
        return cr

    lax.fori_loop(0, nr, rbody, 0)

    _read_hist(h1, hb_v, deg_out, c * (2 * N_NET), s)
    _read_hist(h2, hb_v, deg_out, c * (2 * N_NET) + N_NET, s)

  return _sc_phase2


_TB = 2000


def _tc_x_body(od_ref, nf_ref, x_ref):
    d = jnp.sum(od_ref[...], axis=1, keepdims=True)
    norm = jnp.where(d > 0.0, lax.rsqrt(jnp.maximum(d, 1.0)), 0.0)
    x_ref[...] = nf_ref[...] * norm


def _tc_msg_body(pin_ref, srch_ref, a_ref, c_ref, w2_ref, b2_ref, msg_ref):
    pin = pin_ref[...]
    srch = srch_ref[...]
    pr = jnp.dot(pin, a_ref[...], preferred_element_type=jnp.float32)
    st = jnp.dot(srch, c_ref[...], preferred_element_type=jnp.float32)
    msg_ref[...] = (
        jnp.dot(pr * st, w2_ref[...], preferred_element_type=jnp.float32)
        + jnp.dot(srch, b2_ref[...], preferred_element_type=jnp.float32)
    )


def _tc_final_body(anet_ref, ind_ref, anode_ref, d2_ref, wgc_ref, bgc_ref,
                   bnn_ref, hnet_ref, hnode_ref):
    ind = jnp.sum(ind_ref[...], axis=1, keepdims=True)
    norm = jnp.where(ind > 0.0, lax.rsqrt(jnp.maximum(ind, 1.0)), 0.0)
    anet = (anet_ref[0] + anet_ref[1]) * norm
    hnet_ref[...] = (
        jnp.dot(anet, wgc_ref[...], preferred_element_type=jnp.float32)
        + bgc_ref[...]
    )
    dg = jnp.maximum(jnp.sum(d2_ref[...], axis=1, keepdims=True), 1.0)
    hnode_ref[...] = (anode_ref[0] + anode_ref[1]) / dg + bnn_ref[...]


def kernel(node_feat, net_feat, pin_feat, pins_edge_index, pinned_edge_index,
           W_gc, b_gc, W_lin, b_lin, b_nn):
    idx1 = pins_edge_index.astype(jnp.int32)
    idx2 = pinned_edge_index.astype(jnp.int32)
    src, dst = idx1[0], idx1[1]
    s2, d2 = idx2[0], idx2[1]

    od_flat, src_h = _build_sc_phase1()(src, s2, net_feat)
    od_p = od_flat.reshape(NC, N_NODE)

    x = pl.pallas_call(
        _tc_x_body,
        grid=(N_NODE // _TB,),
        in_specs=[
            pl.BlockSpec((_TB, NC), lambda i: (i, 0)),
            pl.BlockSpec((_TB, H_NODE), lambda i: (i, 0)),
        ],
        out_specs=pl.BlockSpec((_TB, H_NODE), lambda i: (i, 0)),
        out_shape=jax.ShapeDtypeStruct((N_NODE, H_NODE), jnp.float32),
    )(od_p.T, node_feat)

    w2 = W_lin.reshape(H_PIN * H_NET, O_NODE)
    b2 = b_lin.reshape(H_NET, O_NODE)
    eye = jnp.eye(H_PIN, dtype=jnp.float32)
    a_sel = jnp.repeat(eye, H_NET, axis=1)
    c_sel = jnp.tile(eye, (1, H_PIN))
    msg = pl.pallas_call(
        _tc_msg_body,
        grid=(E // _TB,),
        in_specs=[
            pl.BlockSpec((_TB, H_PIN), lambda i: (i, 0)),
            pl.BlockSpec((_TB, H_NET), lambda i: (i, 0)),
            pl.BlockSpec((H_PIN, H_PIN * H_NET), lambda i: (0, 0)),
            pl.BlockSpec((H_NET, H_PIN * H_NET), lambda i: (0, 0)),
            pl.BlockSpec((H_PIN * H_NET, O_NODE), lambda i: (0, 0)),
            pl.BlockSpec((H_NET, O_NODE), lambda i: (0, 0)),
        ],
        out_specs=pl.BlockSpec((_TB, O_NODE), lambda i: (i, 0)),
        out_shape=jax.ShapeDtypeStruct((E, O_NODE), jnp.float32),
    )(pin_feat, src_h, a_sel, c_sel, w2, b2)

    aggnet_f, aggnode_f, deg_flat = _build_sc_phase2()(src, dst, d2, x, msg)
    aggnet_p = aggnet_f.reshape(NC, N_NET, O_NET)
    aggnode_p = aggnode_f.reshape(NC, N_NODE, O_NODE)
    deg_p = deg_flat.reshape(NC, 2, N_NET)

    h_net, h_node = pl.pallas_call(
        _tc_final_body,
        grid=(N_NET // _TB,),
        in_specs=[
            pl.BlockSpec((NC, _TB, O_NET), lambda i: (0, i, 0)),
            pl.BlockSpec((_TB, NC), lambda i: (i, 0)),
            pl.BlockSpec((NC, _TB, O_NODE), lambda i: (0, i, 0)),
            pl.BlockSpec((_TB, NC), lambda i: (i, 0)),
            pl.BlockSpec((H_NODE, O_NET), lambda i: (0, 0)),
            pl.BlockSpec((1, O_NET), lambda i: (0, 0)),
            pl.BlockSpec((1, O_NODE), lambda i: (0, 0)),
        ],
        out_specs=[
            pl.BlockSpec((_TB, O_NET), lambda i: (i, 0)),
            pl.BlockSpec((_TB, O_NODE), lambda i: (i, 0)),
        ],
        out_shape=[
            jax.ShapeDtypeStruct((N_NET, O_NET), jnp.float32),
            jax.ShapeDtypeStruct((N_NODE, O_NODE), jnp.float32),
        ],
    )(aggnet_p, deg_p[:, 0, :].T, aggnode_p, deg_p[:, 1, :].T,
      W_gc, b_gc.reshape(1, O_NET), b_nn.reshape(1, O_NODE))

    return (h_node, h_net)


# speedup vs baseline: 5.4921x; 1.0912x over previous
import functools

import jax
import jax.numpy as jnp
from jax import lax
from jax.experimental import pallas as pl
from jax.experimental.pallas import tpu as pltpu
from jax.experimental.pallas import tpu_sc as plsc

N_NODE = 10000
N_NET = 10000
E = 160000
H_NODE = 128
H_NET = 16
H_PIN = 16
O_NODE = 16
O_NET = 128

NC = 2
NS = 16
NW = NC * NS
EPW = E // NW
CH = 128
NCH = EPW // CH
REM = EPW - NCH * CH
NB = 3
CH2 = 96
NCH2 = EPW // CH2
REM2 = EPW - NCH2 * CH2
RD = 80

ROWS_A = 640
ROWS_B = N_NET - 15 * ROWS_A


def _worker_id():
    c = lax.axis_index("c")
    s = lax.axis_index("s")
    return c * NS + s, c, s


def _init_ones(ones_v):
    one = jnp.full((16,), 1.0, dtype=jnp.float32)
    for i in range(ones_v.shape[0] // 16):
        ones_v[pl.ds(i * 16, 16)] = one


def _fill_zero(ref):
    z = jnp.zeros((16,), dtype=jnp.float32)
    if len(ref.shape) == 1:
        for i in range(ref.shape[0] // 16):
            ref[pl.ds(i * 16, 16)] = z
    else:
        for r in range(ref.shape[0]):
            for j in range(ref.shape[1] // 16):
                ref[r, pl.ds(j * 16, 16)] = z


def _zero_hist(h, zb_v, s):
    base = s * ROWS_A
    nt = jnp.where(s < 15, 8, 5)

    def zbody(t, cr):
        pltpu.sync_copy(zb_v, h.at[pl.ds(base + t * 80, 80)])
        return cr

    lax.fori_loop(0, nt, zbody, 0)


def _read_hist(h, hb_v, out, out_base, s):
    base = s * ROWS_A
    nt = jnp.where(s < 15, 8, 5)

    def rbody(t, cr):
        o = base + t * 80
        pltpu.sync_copy(h.at[pl.ds(o, 80)], hb_v)
        pltpu.sync_copy(hb_v, out.at[pl.ds(pl.multiple_of(out_base + o, 8), 80)])
        return cr

    lax.fori_loop(0, nt, rbody, 0)


@functools.cache
def _build_sc_phase1():
  mesh = plsc.VectorSubcoreMesh(core_axis_name="c", subcore_axis_name="s")

  @functools.partial(
    pl.kernel,
    out_type=(
        jax.ShapeDtypeStruct((NC * N_NODE,), jnp.float32),
        jax.ShapeDtypeStruct((E, H_NET), jnp.float32),
    ),
    mesh=mesh,
    compiler_params=pltpu.CompilerParams(use_tc_tiling_on_sc=False),
    scratch_types=(
        pltpu.VMEM((NB, CH), jnp.int32),
        pltpu.VMEM((NB, CH), jnp.int32),
        pltpu.VMEM((NB, CH, H_NET), jnp.float32),
        pltpu.VMEM((REM,), jnp.int32),
        pltpu.VMEM((REM,), jnp.int32),
        pltpu.VMEM((REM, H_NET), jnp.float32),
        pltpu.VMEM((CH,), jnp.float32),
        pltpu.VMEM((80,), jnp.float32),
        pltpu.VMEM((80,), jnp.float32),
        pltpu.VMEM_SHARED((N_NODE,), jnp.float32),
        pltpu.SemaphoreType.DMA((NB,)),
        pltpu.SemaphoreType.DMA((NB,)),
        pltpu.SemaphoreType.DMA((NB,)),
        pltpu.SemaphoreType.DMA((NB,)),
        pltpu.SemaphoreType.DMA((NB,)),
        pltpu.SemaphoreType.DMA,
    ),
  )
  def _sc_phase1(src_hbm, s2_hbm, net_feat_hbm,
                 od_out, srch_out,
                 isrc, is2, rows, r_isrc, r_is2, r_rows, ones_v, zb_v, hb_v,
                 h0, sem_ls, sem_l2, sem_h, sem_g, sem_w, sem_r):
    wid, c, s = _worker_id()
    _init_ones(ones_v)
    _fill_zero(zb_v)
    _zero_hist(h0, zb_v, s)
    plsc.subcore_barrier()

    e0 = wid * EPW
    d_ls = [None] * NCH
    d_l2 = [None] * NCH
    d_h = [None] * NCH
    d_g = [None] * NCH
    d_w = [None] * NCH

    for t in range(NCH + 2):
        if t < NCH:
            b = t % NB
            if t >= NB:
                d_h[t - NB].wait()
                d_w[t - NB].wait()
            base = e0 + t * CH
            d_ls[t] = pltpu.async_copy(
                src_hbm.at[pl.ds(base, CH)], isrc.at[b], sem_ls.at[b])
            d_l2[t] = pltpu.async_copy(
                s2_hbm.at[pl.ds(base, CH)], is2.at[b], sem_l2.at[b])
        j = t - 1
        if 0 <= j < NCH:
            b = j % NB
            d_ls[j].wait()
            d_h[j] = pltpu.async_copy(ones_v, h0.at[isrc.at[b]], sem_h.at[b],
                                      add=True)
            d_l2[j].wait()
            d_g[j] = pltpu.async_copy(net_feat_hbm.at[is2.at[b]], rows.at[b],
                                      sem_g.at[b])
        j = t - 2
        if j >= 0:
            b = j % NB
            d_g[j].wait()
            d_w[j] = pltpu.async_copy(rows.at[b],
                                      srch_out.at[pl.ds(e0 + j * CH, CH)],
                                      sem_w.at[b])

    for j in range(NCH - NB, NCH):
        d_h[j].wait()
        d_w[j].wait()

    rbase = e0 + NCH * CH
    pltpu.sync_copy(src_hbm.at[pl.ds(rbase, REM)], r_isrc)
    pltpu.sync_copy(ones_v.at[pl.ds(0, REM)], h0.at[r_isrc], add=True)
    pltpu.sync_copy(s2_hbm.at[pl.ds(rbase, REM)], r_is2)
    pltpu.async_copy(net_feat_hbm.at[r_is2], r_rows, sem_r).wait()
    pltpu.sync_copy(r_rows, srch_out.at[pl.ds(rbase, REM)])

    plsc.subcore_barrier()
    _read_hist(h0, hb_v, od_out, c * N_NODE, s)

  return _sc_phase1


@functools.cache
def _build_sc_phase2():
  mesh = plsc.VectorSubcoreMesh(core_axis_name="c", subcore_axis_name="s")

  @functools.partial(
    pl.kernel,
    out_type=(
        jax.ShapeDtypeStruct((NC * N_NET, O_NET), jnp.float32),
        jax.ShapeDtypeStruct((NC * N_NODE, O_NODE), jnp.float32),
        jax.ShapeDtypeStruct((NC * 2 * N_NET,), jnp.float32),
    ),
    mesh=mesh,
    compiler_params=pltpu.CompilerParams(use_tc_tiling_on_sc=False),
    scratch_types=(
        pltpu.VMEM((NB, CH2), jnp.int32),
        pltpu.VMEM((NB, CH2), jnp.int32),
        pltpu.VMEM((NB, CH2), jnp.int32),
        pltpu.VMEM((2, CH2, O_NET), jnp.float32),
        pltpu.VMEM((2, CH2, O_NODE), jnp.float32),
        pltpu.VMEM((REM2,), jnp.int32),
        pltpu.VMEM((REM2,), jnp.int32),
        pltpu.VMEM((REM2,), jnp.int32),
        pltpu.VMEM((REM2, O_NET), jnp.float32),
        pltpu.VMEM((REM2, O_NODE), jnp.float32),
        pltpu.VMEM((CH2,), jnp.float32),
        pltpu.VMEM((8, O_NET), jnp.float32),
        pltpu.VMEM((8, O_NODE), jnp.float32),
        pltpu.VMEM((80,), jnp.float32),
        pltpu.VMEM((80,), jnp.float32),
        pltpu.VMEM_SHARED((N_NET, O_NET), jnp.float32),
        pltpu.VMEM_SHARED((N_NODE, O_NODE), jnp.float32),
        pltpu.VMEM_SHARED((N_NET,), jnp.float32),
        pltpu.VMEM_SHARED((N_NODE,), jnp.float32),
        pltpu.SemaphoreType.DMA((NB,)),
        pltpu.SemaphoreType.DMA((NB,)),
        pltpu.SemaphoreType.DMA((NB,)),
        pltpu.SemaphoreType.DMA((NB,)),
        pltpu.SemaphoreType.DMA((NB,)),
        pltpu.SemaphoreType.DMA((NB,)),
        pltpu.SemaphoreType.DMA((NB,)),
        pltpu.SemaphoreType.DMA((NB,)),
        pltpu.SemaphoreType.DMA((NB,)),
        pltpu.SemaphoreType.DMA,
    ),
  )
  def _sc_phase2(src_hbm, dst_hbm, d2_hbm, x_hbm, msg_hbm,
                 aggnet_out, aggnode_out, deg_out,
                 isrc, idst, id2, rx, rm, r_isrc, r_idst, r_id2, r_rx, r_rm,
                 ones_v, zrow_v, znrow_v, zb_v, hb_v,
                 aggnet_s, aggnode_s, h1, h2,
                 sem_ls, sem_ld, sem_l2, sem_lm, sem_g, sem_sn, sem_sd,
                 sem_h1, sem_h2, sem_r):
    wid, c, s = _worker_id()
    _init_ones(ones_v)
    _fill_zero(zrow_v)
    _fill_zero(znrow_v)
    _fill_zero(zb_v)
    _zero_hist(h1, zb_v, s)
    _zero_hist(h2, zb_v, s)

    base = s * ROWS_A
    nz = jnp.where(s < 15, ROWS_A // 8, ROWS_B // 8)

    def zbody(t, cr):
        pltpu.sync_copy(zrow_v, aggnet_s.at[pl.ds(base + t * 8, 8)])
        pltpu.sync_copy(znrow_v, aggnode_s.at[pl.ds(base + t * 8, 8)])
        return cr

    lax.fori_loop(0, nz, zbody, 0)
    plsc.subcore_barrier()

    e0 = wid * EPW
    d_ls = [None] * NCH2
    d_ld = [None] * NCH2
    d_l2 = [None] * NCH2
    d_lm = [None] * NCH2
    d_g = [None] * NCH2
    d_sn = [None] * NCH2
    d_sd = [None] * NCH2
    d_h1 = [None] * NCH2
    d_h2 = [None] * NCH2

    for t in range(NCH2 + 2):
        j = t - 2
        if j >= 0:
            b = j % NB
            d_g[j].wait()
            d_sn[j] = pltpu.async_copy(rx.at[j % 2], aggnet_s.at[idst.at[b]],
                                       sem_sn.at[b], add=True)
            d_lm[j].wait()
            d_sd[j] = pltpu.async_copy(rm.at[j % 2], aggnode_s.at[id2.at[b]],
                                       sem_sd.at[b], add=True)
        if t < NCH2:
            b = t % NB
            if t >= 2:
                d_sd[t - 2].wait()
            if t >= NB:
                j0 = t - NB
                d_sn[j0].wait()
                d_h1[j0].wait()
                d_h2[j0].wait()
            eb = e0 + t * CH2
            d_ls[t] = pltpu.async_copy(
                src_hbm.at[pl.ds(eb, CH2)], isrc.at[b], sem_ls.at[b])
            d_ld[t] = pltpu.async_copy(
                dst_hbm.at[pl.ds(eb, CH2)], idst.at[b], sem_ld.at[b])
            d_l2[t] = pltpu.async_copy(
                d2_hbm.at[pl.ds(eb, CH2)], id2.at[b], sem_l2.at[b])
            d_lm[t] = pltpu.async_copy(
                msg_hbm.at[pl.ds(eb, CH2)], rm.at[t % 2], sem_lm.at[t % 2])
        j = t - 1
        if 0 <= j < NCH2:
            b = j % NB
            d_ls[j].wait()
            d_g[j] = pltpu.async_copy(x_hbm.at[isrc.at[b]], rx.at[j % 2],
                                      sem_g.at[j % 2])
            d_ld[j].wait()
            d_h1[j] = pltpu.async_copy(ones_v, h1.at[idst.at[b]],
                                       sem_h1.at[b], add=True)
            d_l2[j].wait()
            d_h2[j] = pltpu.async_copy(ones_v, h2.at[id2.at[b]],
                                       sem_h2.at[b], add=True)

    for j in range(NCH2 - NB, NCH2):
        d_sn[j].wait()
        d_h1[j].wait()
        d_h2[j].wait()
    for j in range(NCH2 - 2, NCH2):
        d_sd[j].wait()

    rbase = e0 + NCH2 * CH2
    pltpu.sync_copy(src_hbm.at[pl.ds(rbase, REM2)], r_isrc)
    pltpu.sync_copy(dst_hbm.at[pl.ds(rbase, REM2)], r_idst)
    pltpu.sync_copy(d2_hbm.at[pl.ds(rbase, REM2)], r_id2)
    pltpu.async_copy(x_hbm.at[r_isrc], r_rx, sem_r).wait()
    pltpu.sync_copy(r_rx, aggnet_s.at[r_idst], add=True)
    pltpu.sync_copy(ones_v.at[pl.ds(0, REM2)], h1.at[r_idst], add=True)
    pltpu.sync_copy(msg_hbm.at[pl.ds(rbase, REM2)], r_rm)
    pltpu.sync_copy(r_rm, aggnode_s.at[r_id2], add=True)
    pltpu.sync_copy(ones_v.at[pl.ds(0, REM2)], h2.at[r_id2], add=True)

    plsc.subcore_barrier()
    nr = jnp.where(s < 15, ROWS_A // RD, ROWS_B // RD)

    def rbody(t, cr):
        o = base + t * RD
        oo = pl.multiple_of(c * N_NET + o, 8)
        pltpu.sync_copy(aggnet_s.at[pl.ds(o, RD)], rx.at[0].at[pl.ds(0, RD)])
        pltpu.sync_copy(rx.at[0].at[pl.ds(0, RD)],
                        aggnet_out.at[pl.ds(oo, RD)])
        pltpu.sync_copy(aggnode_s.at[pl.ds(o, RD)], rm.at[0].at[pl.ds(0, RD)])
        pltpu.sync_copy(rm.at[0].at[pl.ds(0, RD)],
                        aggnode_out.at[pl.ds(oo, RD)])
        return cr

    lax.fori_loop(0, nr, rbody, 0)

    _read_hist(h1, hb_v, deg_out, c * (2 * N_NET), s)
    _read_hist(h2, hb_v, deg_out, c * (2 * N_NET) + N_NET, s)

  return _sc_phase2


_TB = 2000
_TBM = 8000


def _tc_x_body(od_ref, nf_ref, x_ref):
    d = jnp.sum(od_ref[...], axis=1, keepdims=True)
    norm = jnp.where(d > 0.0, lax.rsqrt(jnp.maximum(d, 1.0)), 0.0)
    x_ref[...] = nf_ref[...] * norm


def _tc_msg_body(pin_ref, srch_ref, a_ref, c_ref, w2_ref, b2_ref, msg_ref):
    pin = pin_ref[...]
    srch = srch_ref[...]
    pr = jnp.dot(pin, a_ref[...], preferred_element_type=jnp.float32)
    st = jnp.dot(srch, c_ref[...], preferred_element_type=jnp.float32)
    msg_ref[...] = (
        jnp.dot(pr * st, w2_ref[...], preferred_element_type=jnp.float32)
        + jnp.dot(srch, b2_ref[...], preferred_element_type=jnp.float32)
    )


def _tc_final_body(anet_ref, ind_ref, anode_ref, d2_ref, wgc_ref, bgc_ref,
                   bnn_ref, hnet_ref, hnode_ref):
    ind = jnp.sum(ind_ref[...], axis=1, keepdims=True)
    norm = jnp.where(ind > 0.0, lax.rsqrt(jnp.maximum(ind, 1.0)), 0.0)
    anet = (anet_ref[0] + anet_ref[1]) * norm
    hnet_ref[...] = (
        jnp.dot(anet, wgc_ref[...], preferred_element_type=jnp.float32)
        + bgc_ref[...]
    )
    dg = jnp.maximum(jnp.sum(d2_ref[...], axis=1, keepdims=True), 1.0)
    hnode_ref[...] = (anode_ref[0] + anode_ref[1]) / dg + bnn_ref[...]


def kernel(node_feat, net_feat, pin_feat, pins_edge_index, pinned_edge_index,
           W_gc, b_gc, W_lin, b_lin, b_nn):
    idx1 = pins_edge_index.astype(jnp.int32)
    idx2 = pinned_edge_index.astype(jnp.int32)
    src, dst = idx1[0], idx1[1]
    s2, d2 = idx2[0], idx2[1]

    od_flat, src_h = _build_sc_phase1()(src, s2, net_feat)
    od_p = od_flat.reshape(NC, N_NODE)

    x = pl.pallas_call(
        _tc_x_body,
        grid=(N_NODE // _TB,),
        in_specs=[
            pl.BlockSpec((_TB, NC), lambda i: (i, 0)),
            pl.BlockSpec((_TB, H_NODE), lambda i: (i, 0)),
        ],
        out_specs=pl.BlockSpec((_TB, H_NODE), lambda i: (i, 0)),
        out_shape=jax.ShapeDtypeStruct((N_NODE, H_NODE), jnp.float32),
    )(od_p.T, node_feat)

    w2 = W_lin.reshape(H_PIN * H_NET, O_NODE)
    b2 = b_lin.reshape(H_NET, O_NODE)
    eye = jnp.eye(H_PIN, dtype=jnp.float32)
    a_sel = jnp.repeat(eye, H_NET, axis=1)
    c_sel = jnp.tile(eye, (1, H_PIN))
    msg = pl.pallas_call(
        _tc_msg_body,
        grid=(E // _TBM,),
        in_specs=[
            pl.BlockSpec((_TBM, H_PIN), lambda i: (i, 0)),
            pl.BlockSpec((_TBM, H_NET), lambda i: (i, 0)),
            pl.BlockSpec((H_PIN, H_PIN * H_NET), lambda i: (0, 0)),
            pl.BlockSpec((H_NET, H_PIN * H_NET), lambda i: (0, 0)),
            pl.BlockSpec((H_PIN * H_NET, O_NODE), lambda i: (0, 0)),
            pl.BlockSpec((H_NET, O_NODE), lambda i: (0, 0)),
        ],
        out_specs=pl.BlockSpec((_TBM, O_NODE), lambda i: (i, 0)),
        out_shape=jax.ShapeDtypeStruct((E, O_NODE), jnp.float32),
    )(pin_feat, src_h, a_sel, c_sel, w2, b2)

    aggnet_f, aggnode_f, deg_flat = _build_sc_phase2()(src, dst, d2, x, msg)
    aggnet_p = aggnet_f.reshape(NC, N_NET, O_NET)
    aggnode_p = aggnode_f.reshape(NC, N_NODE, O_NODE)
    deg_p = deg_flat.reshape(NC, 2, N_NET)

    h_net, h_node = pl.pallas_call(
        _tc_final_body,
        grid=(N_NET // _TB,),
        in_specs=[
            pl.BlockSpec((NC, _TB, O_NET), lambda i: (0, i, 0)),
            pl.BlockSpec((_TB, NC), lambda i: (i, 0)),
            pl.BlockSpec((NC, _TB, O_NODE), lambda i: (0, i, 0)),
            pl.BlockSpec((_TB, NC), lambda i: (i, 0)),
            pl.BlockSpec((H_NODE, O_NET), lambda i: (0, 0)),
            pl.BlockSpec((1, O_NET), lambda i: (0, 0)),
            pl.BlockSpec((1, O_NODE), lambda i: (0, 0)),
        ],
        out_specs=[
            pl.BlockSpec((_TB, O_NET), lambda i: (i, 0)),
            pl.BlockSpec((_TB, O_NODE), lambda i: (i, 0)),
        ],
        out_shape=[
            jax.ShapeDtypeStruct((N_NET, O_NET), jnp.float32),
            jax.ShapeDtypeStruct((N_NODE, O_NODE), jnp.float32),
        ],
    )(aggnet_p, deg_p[:, 0, :].T, aggnode_p, deg_p[:, 1, :].T,
      W_gc, b_gc.reshape(1, O_NET), b_nn.reshape(1, O_NODE))

    return (h_node, h_net)


# speedup vs baseline: 5.6639x; 1.0313x over previous
import functools

import jax
import jax.numpy as jnp
from jax import lax
from jax.experimental import pallas as pl
from jax.experimental.pallas import tpu as pltpu
from jax.experimental.pallas import tpu_sc as plsc

N_NODE = 10000
N_NET = 10000
E = 160000
H_NODE = 128
H_NET = 16
H_PIN = 16
O_NODE = 16
O_NET = 128

NC = 2
NS = 16
NW = NC * NS
EPW = E // NW
CH = 128
NCH = EPW // CH
REM = EPW - NCH * CH
NB = 3
CH2 = 96
NCH2 = EPW // CH2
REM2 = EPW - NCH2 * CH2
RD = 80

ROWS_A = 640
ROWS_B = N_NET - 15 * ROWS_A


def _worker_id():
    c = lax.axis_index("c")
    s = lax.axis_index("s")
    return c * NS + s, c, s


def _init_ones(ones_v):
    one = jnp.full((16,), 1.0, dtype=jnp.float32)
    for i in range(ones_v.shape[0] // 16):
        ones_v[pl.ds(i * 16, 16)] = one


def _fill_zero(ref):
    z = jnp.zeros((16,), dtype=jnp.float32)
    if len(ref.shape) == 1:
        for i in range(ref.shape[0] // 16):
            ref[pl.ds(i * 16, 16)] = z
    else:
        for r in range(ref.shape[0]):
            for j in range(ref.shape[1] // 16):
                ref[r, pl.ds(j * 16, 16)] = z


def _zero_hist(h, zb_v, s):
    base = s * ROWS_A
    nt = jnp.where(s < 15, 8, 5)

    def zbody(t, cr):
        pltpu.sync_copy(zb_v, h.at[pl.ds(base + t * 80, 80)])
        return cr

    lax.fori_loop(0, nt, zbody, 0)


def _read_hist(h, hb_v, out, out_base, s):
    base = s * ROWS_A
    nt = jnp.where(s < 15, 8, 5)

    def rbody(t, cr):
        o = base + t * 80
        pltpu.sync_copy(h.at[pl.ds(o, 80)], hb_v)
        pltpu.sync_copy(hb_v, out.at[pl.ds(pl.multiple_of(out_base + o, 8), 80)])
        return cr

    lax.fori_loop(0, nt, rbody, 0)


@functools.cache
def _build_sc_phase1():
  mesh = plsc.VectorSubcoreMesh(core_axis_name="c", subcore_axis_name="s")

  @functools.partial(
    pl.kernel,
    out_type=(
        jax.ShapeDtypeStruct((NC * N_NODE,), jnp.float32),
        jax.ShapeDtypeStruct((E, H_NET), jnp.float32),
    ),
    mesh=mesh,
    compiler_params=pltpu.CompilerParams(use_tc_tiling_on_sc=False),
    scratch_types=(
        pltpu.VMEM((NB, CH), jnp.int32),
        pltpu.VMEM((NB, CH), jnp.int32),
        pltpu.VMEM((NB, CH, H_NET), jnp.float32),
        pltpu.VMEM((REM,), jnp.int32),
        pltpu.VMEM((REM,), jnp.int32),
        pltpu.VMEM((REM, H_NET), jnp.float32),
        pltpu.VMEM((CH,), jnp.float32),
        pltpu.VMEM((80,), jnp.float32),
        pltpu.VMEM((80,), jnp.float32),
        pltpu.VMEM_SHARED((N_NODE,), jnp.float32),
        pltpu.SemaphoreType.DMA((NB,)),
        pltpu.SemaphoreType.DMA((NB,)),
        pltpu.SemaphoreType.DMA((NB,)),
        pltpu.SemaphoreType.DMA((NB,)),
        pltpu.SemaphoreType.DMA((NB,)),
        pltpu.SemaphoreType.DMA,
    ),
  )
  def _sc_phase1(src_hbm, s2_hbm, net_feat_hbm,
                 od_out, srch_out,
                 isrc, is2, rows, r_isrc, r_is2, r_rows, ones_v, zb_v, hb_v,
                 h0, sem_ls, sem_l2, sem_h, sem_g, sem_w, sem_r):
    wid, c, s = _worker_id()
    _init_ones(ones_v)
    _fill_zero(zb_v)
    _zero_hist(h0, zb_v, s)
    plsc.subcore_barrier()

    e0 = wid * EPW
    d_ls = [None] * NCH
    d_l2 = [None] * NCH
    d_h = [None] * NCH
    d_g = [None] * NCH
    d_w = [None] * NCH

    for t in range(NCH + 2):
        if t < NCH:
            b = t % NB
            if t >= NB:
                d_h[t - NB].wait()
                d_w[t - NB].wait()
            base = e0 + t * CH
            d_ls[t] = pltpu.async_copy(
                src_hbm.at[pl.ds(base, CH)], isrc.at[b], sem_ls.at[b])
            d_l2[t] = pltpu.async_copy(
                s2_hbm.at[pl.ds(base, CH)], is2.at[b], sem_l2.at[b])
        j = t - 1
        if 0 <= j < NCH:
            b = j % NB
            d_ls[j].wait()
            d_h[j] = pltpu.async_copy(ones_v, h0.at[isrc.at[b]], sem_h.at[b],
                                      add=True)
            d_l2[j].wait()
            d_g[j] = pltpu.async_copy(net_feat_hbm.at[is2.at[b]], rows.at[b],
                                      sem_g.at[b])
        j = t - 2
        if j >= 0:
            b = j % NB
            d_g[j].wait()
            d_w[j] = pltpu.async_copy(rows.at[b],
                                      srch_out.at[pl.ds(e0 + j * CH, CH)],
                                      sem_w.at[b])

    for j in range(NCH - NB, NCH):
        d_h[j].wait()
        d_w[j].wait()

    rbase = e0 + NCH * CH
    pltpu.sync_copy(src_hbm.at[pl.ds(rbase, REM)], r_isrc)
    pltpu.sync_copy(ones_v.at[pl.ds(0, REM)], h0.at[r_isrc], add=True)
    pltpu.sync_copy(s2_hbm.at[pl.ds(rbase, REM)], r_is2)
    pltpu.async_copy(net_feat_hbm.at[r_is2], r_rows, sem_r).wait()
    pltpu.sync_copy(r_rows, srch_out.at[pl.ds(rbase, REM)])

    plsc.subcore_barrier()
    _read_hist(h0, hb_v, od_out, c * N_NODE, s)

  return _sc_phase1


@functools.cache
def _build_sc_phase2():
  mesh = plsc.VectorSubcoreMesh(core_axis_name="c", subcore_axis_name="s")

  @functools.partial(
    pl.kernel,
    out_type=(
        jax.ShapeDtypeStruct((NC * N_NET, O_NET), jnp.float32),
        jax.ShapeDtypeStruct((NC * N_NODE, O_NODE), jnp.float32),
        jax.ShapeDtypeStruct((NC * 2 * N_NET,), jnp.float32),
    ),
    mesh=mesh,
    compiler_params=pltpu.CompilerParams(use_tc_tiling_on_sc=False),
    scratch_types=(
        pltpu.VMEM((NB, CH2), jnp.int32),
        pltpu.VMEM((NB, CH2), jnp.int32),
        pltpu.VMEM((NB, CH2), jnp.int32),
        pltpu.VMEM((2, CH2, O_NET), jnp.float32),
        pltpu.VMEM((2, CH2, O_NODE), jnp.float32),
        pltpu.VMEM((REM2,), jnp.int32),
        pltpu.VMEM((REM2,), jnp.int32),
        pltpu.VMEM((REM2,), jnp.int32),
        pltpu.VMEM((REM2, O_NET), jnp.float32),
        pltpu.VMEM((REM2, O_NODE), jnp.float32),
        pltpu.VMEM((CH2,), jnp.float32),
        pltpu.VMEM((8, O_NET), jnp.float32),
        pltpu.VMEM((8, O_NODE), jnp.float32),
        pltpu.VMEM((80,), jnp.float32),
        pltpu.VMEM((80,), jnp.float32),
        pltpu.VMEM_SHARED((N_NET, O_NET), jnp.float32),
        pltpu.VMEM_SHARED((N_NODE, O_NODE), jnp.float32),
        pltpu.VMEM_SHARED((N_NET,), jnp.float32),
        pltpu.VMEM_SHARED((N_NODE,), jnp.float32),
        pltpu.SemaphoreType.DMA((NB,)),
        pltpu.SemaphoreType.DMA((NB,)),
        pltpu.SemaphoreType.DMA((NB,)),
        pltpu.SemaphoreType.DMA((NB,)),
        pltpu.SemaphoreType.DMA((NB,)),
        pltpu.SemaphoreType.DMA((NB,)),
        pltpu.SemaphoreType.DMA((NB,)),
        pltpu.SemaphoreType.DMA((NB,)),
        pltpu.SemaphoreType.DMA((NB,)),
        pltpu.SemaphoreType.DMA,
    ),
  )
  def _sc_phase2(src_hbm, dst_hbm, d2_hbm, x_hbm, msg_hbm,
                 aggnet_out, aggnode_out, deg_out,
                 isrc, idst, id2, rx, rm, r_isrc, r_idst, r_id2, r_rx, r_rm,
                 ones_v, zrow_v, znrow_v, zb_v, hb_v,
                 aggnet_s, aggnode_s, h1, h2,
                 sem_ls, sem_ld, sem_l2, sem_lm, sem_g, sem_sn, sem_sd,
                 sem_h1, sem_h2, sem_r):
    wid, c, s = _worker_id()
    _init_ones(ones_v)
    _fill_zero(zrow_v)
    _fill_zero(znrow_v)
    _fill_zero(zb_v)
    _zero_hist(h1, zb_v, s)
    _zero_hist(h2, zb_v, s)

    base = s * ROWS_A
    nz = jnp.where(s < 15, ROWS_A // 8, ROWS_B // 8)

    def zbody(t, cr):
        pltpu.sync_copy(zrow_v, aggnet_s.at[pl.ds(base + t * 8, 8)])
        pltpu.sync_copy(znrow_v, aggnode_s.at[pl.ds(base + t * 8, 8)])
        return cr

    lax.fori_loop(0, nz, zbody, 0)
    plsc.subcore_barrier()

    e0 = wid * EPW
    d_ls = [None] * NCH2
    d_ld = [None] * NCH2
    d_l2 = [None] * NCH2
    d_lm = [None] * NCH2
    d_g = [None] * NCH2
    d_sn = [None] * NCH2
    d_sd = [None] * NCH2
    d_h1 = [None] * NCH2
    d_h2 = [None] * NCH2

    for t in range(NCH2 + 2):
        j = t - 2
        if j >= 0:
            b = j % NB
            d_g[j].wait()
            d_sn[j] = pltpu.async_copy(rx.at[j % 2], aggnet_s.at[idst.at[b]],
                                       sem_sn.at[b], add=True)
            d_lm[j].wait()
            d_sd[j] = pltpu.async_copy(rm.at[j % 2], aggnode_s.at[id2.at[b]],
                                       sem_sd.at[b], add=True)
        if t < NCH2:
            b = t % NB
            if t >= 2:
                d_sd[t - 2].wait()
            if t >= NB:
                j0 = t - NB
                d_sn[j0].wait()
                d_h1[j0].wait()
                d_h2[j0].wait()
            eb = e0 + t * CH2
            d_ls[t] = pltpu.async_copy(
                src_hbm.at[pl.ds(eb, CH2)], isrc.at[b], sem_ls.at[b])
            d_ld[t] = pltpu.async_copy(
                dst_hbm.at[pl.ds(eb, CH2)], idst.at[b], sem_ld.at[b])
            d_l2[t] = pltpu.async_copy(
                d2_hbm.at[pl.ds(eb, CH2)], id2.at[b], sem_l2.at[b])
            d_lm[t] = pltpu.async_copy(
                msg_hbm.at[pl.ds(eb, CH2)], rm.at[t % 2], sem_lm.at[t % 2])
        j = t - 1
        if 0 <= j < NCH2:
            b = j % NB
            d_ls[j].wait()
            d_g[j] = pltpu.async_copy(x_hbm.at[isrc.at[b]], rx.at[j % 2],
                                      sem_g.at[j % 2])
            d_ld[j].wait()
            d_h1[j] = pltpu.async_copy(ones_v, h1.at[idst.at[b]],
                                       sem_h1.at[b], add=True)
            d_l2[j].wait()
            d_h2[j] = pltpu.async_copy(ones_v, h2.at[id2.at[b]],
                                       sem_h2.at[b], add=True)

    for j in range(NCH2 - NB, NCH2):
        d_sn[j].wait()
        d_h1[j].wait()
        d_h2[j].wait()
    for j in range(NCH2 - 2, NCH2):
        d_sd[j].wait()

    rbase = e0 + NCH2 * CH2
    pltpu.sync_copy(src_hbm.at[pl.ds(rbase, REM2)], r_isrc)
    pltpu.sync_copy(dst_hbm.at[pl.ds(rbase, REM2)], r_idst)
    pltpu.sync_copy(d2_hbm.at[pl.ds(rbase, REM2)], r_id2)
    pltpu.async_copy(x_hbm.at[r_isrc], r_rx, sem_r).wait()
    pltpu.sync_copy(r_rx, aggnet_s.at[r_idst], add=True)
    pltpu.sync_copy(ones_v.at[pl.ds(0, REM2)], h1.at[r_idst], add=True)
    pltpu.sync_copy(msg_hbm.at[pl.ds(rbase, REM2)], r_rm)
    pltpu.sync_copy(r_rm, aggnode_s.at[r_id2], add=True)
    pltpu.sync_copy(ones_v.at[pl.ds(0, REM2)], h2.at[r_id2], add=True)

    plsc.subcore_barrier()
    nr = jnp.where(s < 15, ROWS_A // RD, ROWS_B // RD)

    def rbody(t, cr):
        o = base + t * RD
        oo = pl.multiple_of(c * N_NET + o, 8)
        pltpu.sync_copy(aggnet_s.at[pl.ds(o, RD)], rx.at[0].at[pl.ds(0, RD)])
        pltpu.sync_copy(rx.at[0].at[pl.ds(0, RD)],
                        aggnet_out.at[pl.ds(oo, RD)])
        pltpu.sync_copy(aggnode_s.at[pl.ds(o, RD)], rm.at[0].at[pl.ds(0, RD)])
        pltpu.sync_copy(rm.at[0].at[pl.ds(0, RD)],
                        aggnode_out.at[pl.ds(oo, RD)])
        return cr

    lax.fori_loop(0, nr, rbody, 0)

    _read_hist(h1, hb_v, deg_out, c * (2 * N_NET), s)
    _read_hist(h2, hb_v, deg_out, c * (2 * N_NET) + N_NET, s)

  return _sc_phase2


_TB = 2000
_TBM = 8000


def _tc_x_body(od_ref, nf_ref, x_ref):
    d = (od_ref[0] + od_ref[1]).reshape(-1, 1)
    norm = jnp.where(d > 0.0, lax.rsqrt(jnp.maximum(d, 1.0)), 0.0)
    x_ref[...] = nf_ref[...] * norm


def _tc_msg_body(pin_ref, srch_ref, a_ref, c_ref, w2_ref, b2_ref, msg_ref):
    pin = pin_ref[...]
    srch = srch_ref[...]
    pr = jnp.dot(pin, a_ref[...], preferred_element_type=jnp.float32)
    st = jnp.dot(srch, c_ref[...], preferred_element_type=jnp.float32)
    msg_ref[...] = (
        jnp.dot(pr * st, w2_ref[...], preferred_element_type=jnp.float32)
        + jnp.dot(srch, b2_ref[...], preferred_element_type=jnp.float32)
    )


def _tc_final_body(anet_ref, deg_ref, anode_ref, wgc_ref, bgc_ref,
                   bnn_ref, hnet_ref, hnode_ref):
    ind = (deg_ref[0] + deg_ref[2]).reshape(-1, 1)
    norm = jnp.where(ind > 0.0, lax.rsqrt(jnp.maximum(ind, 1.0)), 0.0)
    anet = (anet_ref[0] + anet_ref[1]) * norm
    hnet_ref[...] = (
        jnp.dot(anet, wgc_ref[...], preferred_element_type=jnp.float32)
        + bgc_ref[...]
    )
    dg = jnp.maximum((deg_ref[1] + deg_ref[3]).reshape(-1, 1), 1.0)
    hnode_ref[...] = (anode_ref[0] + anode_ref[1]) / dg + bnn_ref[...]


def kernel(node_feat, net_feat, pin_feat, pins_edge_index, pinned_edge_index,
           W_gc, b_gc, W_lin, b_lin, b_nn):
    idx1 = pins_edge_index.astype(jnp.int32)
    idx2 = pinned_edge_index.astype(jnp.int32)
    src, dst = idx1[0], idx1[1]
    s2, d2 = idx2[0], idx2[1]

    od_flat, src_h = _build_sc_phase1()(src, s2, net_feat)

    x = pl.pallas_call(
        _tc_x_body,
        out_shape=jax.ShapeDtypeStruct((N_NODE, H_NODE), jnp.float32),
    )(od_flat.reshape(NC, N_NODE), node_feat)

    w2 = W_lin.reshape(H_PIN * H_NET, O_NODE)
    b2 = b_lin.reshape(H_NET, O_NODE)
    eye = jnp.eye(H_PIN, dtype=jnp.float32)
    a_sel = jnp.repeat(eye, H_NET, axis=1)
    c_sel = jnp.tile(eye, (1, H_PIN))
    msg = pl.pallas_call(
        _tc_msg_body,
        grid=(E // _TBM,),
        in_specs=[
            pl.BlockSpec((_TBM, H_PIN), lambda i: (i, 0)),
            pl.BlockSpec((_TBM, H_NET), lambda i: (i, 0)),
            pl.BlockSpec((H_PIN, H_PIN * H_NET), lambda i: (0, 0)),
            pl.BlockSpec((H_NET, H_PIN * H_NET), lambda i: (0, 0)),
            pl.BlockSpec((H_PIN * H_NET, O_NODE), lambda i: (0, 0)),
            pl.BlockSpec((H_NET, O_NODE), lambda i: (0, 0)),
        ],
        out_specs=pl.BlockSpec((_TBM, O_NODE), lambda i: (i, 0)),
        out_shape=jax.ShapeDtypeStruct((E, O_NODE), jnp.float32),
    )(pin_feat, src_h, a_sel, c_sel, w2, b2)

    aggnet_f, aggnode_f, deg_flat = _build_sc_phase2()(src, dst, d2, x, msg)
    aggnet_p = aggnet_f.reshape(NC, N_NET, O_NET)
    aggnode_p = aggnode_f.reshape(NC, N_NODE, O_NODE)

    h_net, h_node = pl.pallas_call(
        _tc_final_body,
        out_shape=[
            jax.ShapeDtypeStruct((N_NET, O_NET), jnp.float32),
            jax.ShapeDtypeStruct((N_NODE, O_NODE), jnp.float32),
        ],
    )(aggnet_p, deg_flat.reshape(NC * 2, N_NET), aggnode_p,
      W_gc, b_gc.reshape(1, O_NET), b_nn.reshape(1, O_NODE))

    return (h_node, h_net)


# speedup vs baseline: 5.7414x; 1.0137x over previous
import functools

import jax
import jax.numpy as jnp
from jax import lax
from jax.experimental import pallas as pl
from jax.experimental.pallas import tpu as pltpu
from jax.experimental.pallas import tpu_sc as plsc

N_NODE = 10000
N_NET = 10000
E = 160000
H_NODE = 128
H_NET = 16
H_PIN = 16
O_NODE = 16
O_NET = 128

NC = 2
NS = 16
NW = NC * NS
EPW = E // NW
CH = 256
NCH = EPW // CH
REM = EPW - NCH * CH
NB = 3
CH2 = 96
NCH2 = EPW // CH2
REM2 = EPW - NCH2 * CH2
RD = 80

ROWS_A = 640
ROWS_B = N_NET - 15 * ROWS_A


def _worker_id():
    c = lax.axis_index("c")
    s = lax.axis_index("s")
    return c * NS + s, c, s


def _init_ones(ones_v):
    one = jnp.full((16,), 1.0, dtype=jnp.float32)
    for i in range(ones_v.shape[0] // 16):
        ones_v[pl.ds(i * 16, 16)] = one


def _fill_zero(ref):
    z = jnp.zeros((16,), dtype=jnp.float32)
    if len(ref.shape) == 1:
        for i in range(ref.shape[0] // 16):
            ref[pl.ds(i * 16, 16)] = z
    else:
        for r in range(ref.shape[0]):
            for j in range(ref.shape[1] // 16):
                ref[r, pl.ds(j * 16, 16)] = z


def _zero_hist(h, zb_v, s):
    base = s * ROWS_A
    nt = jnp.where(s < 15, 8, 5)

    def zbody(t, cr):
        pltpu.sync_copy(zb_v, h.at[pl.ds(base + t * 80, 80)])
        return cr

    lax.fori_loop(0, nt, zbody, 0)


def _read_hist(h, hb_v, out, out_base, s):
    base = s * ROWS_A
    nt = jnp.where(s < 15, 8, 5)

    def rbody(t, cr):
        o = base + t * 80
        pltpu.sync_copy(h.at[pl.ds(o, 80)], hb_v)
        pltpu.sync_copy(hb_v, out.at[pl.ds(pl.multiple_of(out_base + o, 8), 80)])
        return cr

    lax.fori_loop(0, nt, rbody, 0)


@functools.cache
def _build_sc_phase1():
  mesh = plsc.VectorSubcoreMesh(core_axis_name="c", subcore_axis_name="s")

  @functools.partial(
    pl.kernel,
    out_type=(
        jax.ShapeDtypeStruct((NC * N_NODE,), jnp.float32),
        jax.ShapeDtypeStruct((E, H_NET), jnp.float32),
    ),
    mesh=mesh,
    compiler_params=pltpu.CompilerParams(use_tc_tiling_on_sc=False),
    scratch_types=(
        pltpu.VMEM((NB, CH), jnp.int32),
        pltpu.VMEM((NB, CH), jnp.int32),
        pltpu.VMEM((NB, CH, H_NET), jnp.float32),
        pltpu.VMEM((REM,), jnp.int32),
        pltpu.VMEM((REM,), jnp.int32),
        pltpu.VMEM((REM, H_NET), jnp.float32),
        pltpu.VMEM((CH,), jnp.float32),
        pltpu.VMEM((80,), jnp.float32),
        pltpu.VMEM((80,), jnp.float32),
        pltpu.VMEM_SHARED((N_NODE,), jnp.float32),
        pltpu.SemaphoreType.DMA((NB,)),
        pltpu.SemaphoreType.DMA((NB,)),
        pltpu.SemaphoreType.DMA((NB,)),
        pltpu.SemaphoreType.DMA((NB,)),
        pltpu.SemaphoreType.DMA((NB,)),
        pltpu.SemaphoreType.DMA,
    ),
  )
  def _sc_phase1(src_hbm, s2_hbm, net_feat_hbm,
                 od_out, srch_out,
                 isrc, is2, rows, r_isrc, r_is2, r_rows, ones_v, zb_v, hb_v,
                 h0, sem_ls, sem_l2, sem_h, sem_g, sem_w, sem_r):
    wid, c, s = _worker_id()
    _init_ones(ones_v)
    _fill_zero(zb_v)
    _zero_hist(h0, zb_v, s)
    plsc.subcore_barrier()

    e0 = wid * EPW
    d_ls = [None] * NCH
    d_l2 = [None] * NCH
    d_h = [None] * NCH
    d_g = [None] * NCH
    d_w = [None] * NCH

    for t in range(NCH + 2):
        if t < NCH:
            b = t % NB
            if t >= NB:
                d_h[t - NB].wait()
                d_w[t - NB].wait()
            base = e0 + t * CH
            d_ls[t] = pltpu.async_copy(
                src_hbm.at[pl.ds(base, CH)], isrc.at[b], sem_ls.at[b])
            d_l2[t] = pltpu.async_copy(
                s2_hbm.at[pl.ds(base, CH)], is2.at[b], sem_l2.at[b])
        j = t - 1
        if 0 <= j < NCH:
            b = j % NB
            d_ls[j].wait()
            d_h[j] = pltpu.async_copy(ones_v, h0.at[isrc.at[b]], sem_h.at[b],
                                      add=True)
            d_l2[j].wait()
            d_g[j] = pltpu.async_copy(net_feat_hbm.at[is2.at[b]], rows.at[b],
                                      sem_g.at[b])
        j = t - 2
        if j >= 0:
            b = j % NB
            d_g[j].wait()
            d_w[j] = pltpu.async_copy(rows.at[b],
                                      srch_out.at[pl.ds(e0 + j * CH, CH)],
                                      sem_w.at[b])

    for j in range(NCH - NB, NCH):
        d_h[j].wait()
        d_w[j].wait()

    rbase = e0 + NCH * CH
    pltpu.sync_copy(src_hbm.at[pl.ds(rbase, REM)], r_isrc)
    pltpu.sync_copy(ones_v.at[pl.ds(0, REM)], h0.at[r_isrc], add=True)
    pltpu.sync_copy(s2_hbm.at[pl.ds(rbase, REM)], r_is2)
    pltpu.async_copy(net_feat_hbm.at[r_is2], r_rows, sem_r).wait()
    pltpu.sync_copy(r_rows, srch_out.at[pl.ds(rbase, REM)])

    plsc.subcore_barrier()
    _read_hist(h0, hb_v, od_out, c * N_NODE, s)

  return _sc_phase1


@functools.cache
def _build_sc_phase2():
  mesh = plsc.VectorSubcoreMesh(core_axis_name="c", subcore_axis_name="s")

  @functools.partial(
    pl.kernel,
    out_type=(
        jax.ShapeDtypeStruct((NC * N_NET, O_NET), jnp.float32),
        jax.ShapeDtypeStruct((NC * N_NODE, O_NODE), jnp.float32),
        jax.ShapeDtypeStruct((NC * 2 * N_NET,), jnp.float32),
    ),
    mesh=mesh,
    compiler_params=pltpu.CompilerParams(use_tc_tiling_on_sc=False),
    scratch_types=(
        pltpu.VMEM((NB, CH2), jnp.int32),
        pltpu.VMEM((NB, CH2), jnp.int32),
        pltpu.VMEM((NB, CH2), jnp.int32),
        pltpu.VMEM((2, CH2, O_NET), jnp.float32),
        pltpu.VMEM((2, CH2, O_NODE), jnp.float32),
        pltpu.VMEM((REM2,), jnp.int32),
        pltpu.VMEM((REM2,), jnp.int32),
        pltpu.VMEM((REM2,), jnp.int32),
        pltpu.VMEM((REM2, O_NET), jnp.float32),
        pltpu.VMEM((REM2, O_NODE), jnp.float32),
        pltpu.VMEM((CH2,), jnp.float32),
        pltpu.VMEM((8, O_NET), jnp.float32),
        pltpu.VMEM((8, O_NODE), jnp.float32),
        pltpu.VMEM((80,), jnp.float32),
        pltpu.VMEM((80,), jnp.float32),
        pltpu.VMEM_SHARED((N_NET, O_NET), jnp.float32),
        pltpu.VMEM_SHARED((N_NODE, O_NODE), jnp.float32),
        pltpu.VMEM_SHARED((N_NET,), jnp.float32),
        pltpu.VMEM_SHARED((N_NODE,), jnp.float32),
        pltpu.SemaphoreType.DMA((NB,)),
        pltpu.SemaphoreType.DMA((NB,)),
        pltpu.SemaphoreType.DMA((NB,)),
        pltpu.SemaphoreType.DMA((NB,)),
        pltpu.SemaphoreType.DMA((NB,)),
        pltpu.SemaphoreType.DMA((NB,)),
        pltpu.SemaphoreType.DMA((NB,)),
        pltpu.SemaphoreType.DMA((NB,)),
        pltpu.SemaphoreType.DMA((NB,)),
        pltpu.SemaphoreType.DMA,
    ),
  )
  def _sc_phase2(src_hbm, dst_hbm, d2_hbm, x_hbm, msg_hbm,
                 aggnet_out, aggnode_out, deg_out,
                 isrc, idst, id2, rx, rm, r_isrc, r_idst, r_id2, r_rx, r_rm,
                 ones_v, zrow_v, znrow_v, zb_v, hb_v,
                 aggnet_s, aggnode_s, h1, h2,
                 sem_ls, sem_ld, sem_l2, sem_lm, sem_g, sem_sn, sem_sd,
                 sem_h1, sem_h2, sem_r):
    wid, c, s = _worker_id()
    _init_ones(ones_v)
    _fill_zero(zrow_v)
    _fill_zero(znrow_v)
    _fill_zero(zb_v)
    _zero_hist(h1, zb_v, s)
    _zero_hist(h2, zb_v, s)

    _fill_zero(rx.at[0])
    _fill_zero(rm.at[0])
    base = s * ROWS_A
    nz = jnp.where(s < 15, ROWS_A // RD, ROWS_B // RD)

    def zbody(t, cr):
        pltpu.sync_copy(rx.at[0].at[pl.ds(0, RD)],
                        aggnet_s.at[pl.ds(base + t * RD, RD)])
        pltpu.sync_copy(rm.at[0].at[pl.ds(0, RD)],
                        aggnode_s.at[pl.ds(base + t * RD, RD)])
        return cr

    lax.fori_loop(0, nz, zbody, 0)
    plsc.subcore_barrier()

    e0 = wid * EPW
    d_ls = [None] * NCH2
    d_ld = [None] * NCH2
    d_l2 = [None] * NCH2
    d_lm = [None] * NCH2
    d_g = [None] * NCH2
    d_sn = [None] * NCH2
    d_sd = [None] * NCH2
    d_h1 = [None] * NCH2
    d_h2 = [None] * NCH2

    for t in range(NCH2 + 2):
        j = t - 2
        if j >= 0:
            b = j % NB
            d_g[j].wait()
            d_sn[j] = pltpu.async_copy(rx.at[j % 2], aggnet_s.at[idst.at[b]],
                                       sem_sn.at[b], add=True)
            d_lm[j].wait()
            d_sd[j] = pltpu.async_copy(rm.at[j % 2], aggnode_s.at[id2.at[b]],
                                       sem_sd.at[b], add=True)
        if t < NCH2:
            b = t % NB
            if t >= 2:
                d_sd[t - 2].wait()
            if t >= NB:
                j0 = t - NB
                d_sn[j0].wait()
                d_h1[j0].wait()
                d_h2[j0].wait()
            eb = e0 + t * CH2
            d_ls[t] = pltpu.async_copy(
                src_hbm.at[pl.ds(eb, CH2)], isrc.at[b], sem_ls.at[b])
            d_ld[t] = pltpu.async_copy(
                dst_hbm.at[pl.ds(eb, CH2)], idst.at[b], sem_ld.at[b])
            d_l2[t] = pltpu.async_copy(
                d2_hbm.at[pl.ds(eb, CH2)], id2.at[b], sem_l2.at[b])
            d_lm[t] = pltpu.async_copy(
                msg_hbm.at[pl.ds(eb, CH2)], rm.at[t % 2], sem_lm.at[t % 2])
        j = t - 1
        if 0 <= j < NCH2:
            b = j % NB
            d_ls[j].wait()
            d_g[j] = pltpu.async_copy(x_hbm.at[isrc.at[b]], rx.at[j % 2],
                                      sem_g.at[j % 2])
            d_ld[j].wait()
            d_h1[j] = pltpu.async_copy(ones_v, h1.at[idst.at[b]],
                                       sem_h1.at[b], add=True)
            d_l2[j].wait()
            d_h2[j] = pltpu.async_copy(ones_v, h2.at[id2.at[b]],
                                       sem_h2.at[b], add=True)

    for j in range(NCH2 - NB, NCH2):
        d_sn[j].wait()
        d_h1[j].wait()
        d_h2[j].wait()
    for j in range(NCH2 - 2, NCH2):
        d_sd[j].wait()

    rbase = e0 + NCH2 * CH2
    pltpu.sync_copy(src_hbm.at[pl.ds(rbase, REM2)], r_isrc)
    pltpu.sync_copy(dst_hbm.at[pl.ds(rbase, REM2)], r_idst)
    pltpu.sync_copy(d2_hbm.at[pl.ds(rbase, REM2)], r_id2)
    pltpu.async_copy(x_hbm.at[r_isrc], r_rx, sem_r).wait()
    pltpu.sync_copy(r_rx, aggnet_s.at[r_idst], add=True)
    pltpu.sync_copy(ones_v.at[pl.ds(0, REM2)], h1.at[r_idst], add=True)
    pltpu.sync_copy(msg_hbm.at[pl.ds(rbase, REM2)], r_rm)
    pltpu.sync_copy(r_rm, aggnode_s.at[r_id2], add=True)
    pltpu.sync_copy(ones_v.at[pl.ds(0, REM2)], h2.at[r_id2], add=True)

    plsc.subcore_barrier()

    def emit_readout(nchunks):
        hn = [None] * nchunks
        hd = [None] * nchunks
        for t in range(nchunks):
            b = t % 2
            if t >= 2:
                hn[t - 2].wait()
                hd[t - 2].wait()
            o = base + t * RD
            oo = pl.multiple_of(c * N_NET + o, 8)
            pltpu.sync_copy(aggnet_s.at[pl.ds(o, RD)],
                            rx.at[b].at[pl.ds(0, RD)])
            hn[t] = pltpu.async_copy(rx.at[b].at[pl.ds(0, RD)],
                                     aggnet_out.at[pl.ds(oo, RD)],
                                     sem_ls.at[b])
            pltpu.sync_copy(aggnode_s.at[pl.ds(o, RD)],
                            rm.at[b].at[pl.ds(0, RD)])
            hd[t] = pltpu.async_copy(rm.at[b].at[pl.ds(0, RD)],
                                     aggnode_out.at[pl.ds(oo, RD)],
                                     sem_ld.at[b])
        for t in range(max(0, nchunks - 2), nchunks):
            hn[t].wait()
            hd[t].wait()

    @pl.when(s < 15)
    def _():
        emit_readout(ROWS_A // RD)

    @pl.when(s == 15)
    def _():
        emit_readout(ROWS_B // RD)

    _read_hist(h1, hb_v, deg_out, c * (2 * N_NET), s)
    _read_hist(h2, hb_v, deg_out, c * (2 * N_NET) + N_NET, s)

  return _sc_phase2


_TB = 2000
_TBM = 8000


def _tc_x_body(od_ref, nf_ref, x_ref):
    d = (od_ref[0] + od_ref[1]).reshape(-1, 1)
    norm = jnp.where(d > 0.0, lax.rsqrt(jnp.maximum(d, 1.0)), 0.0)
    x_ref[...] = nf_ref[...] * norm


def _tc_msg_body(pin_ref, srch_ref, a_ref, c_ref, w2_ref, b2_ref, msg_ref):
    pin = pin_ref[...]
    srch = srch_ref[...]
    pr = jnp.dot(pin, a_ref[...], preferred_element_type=jnp.float32)
    st = jnp.dot(srch, c_ref[...], preferred_element_type=jnp.float32)
    msg_ref[...] = (
        jnp.dot(pr * st, w2_ref[...], preferred_element_type=jnp.float32)
        + jnp.dot(srch, b2_ref[...], preferred_element_type=jnp.float32)
    )


def _tc_final_body(anet_ref, deg_ref, anode_ref, wgc_ref, bgc_ref,
                   bnn_ref, hnet_ref, hnode_ref):
    ind = (deg_ref[0] + deg_ref[2]).reshape(-1, 1)
    norm = jnp.where(ind > 0.0, lax.rsqrt(jnp.maximum(ind, 1.0)), 0.0)
    anet = (anet_ref[0] + anet_ref[1]) * norm
    hnet_ref[...] = (
        jnp.dot(anet, wgc_ref[...], preferred_element_type=jnp.float32)
        + bgc_ref[...]
    )
    dg = jnp.maximum((deg_ref[1] + deg_ref[3]).reshape(-1, 1), 1.0)
    hnode_ref[...] = (anode_ref[0] + anode_ref[1]) / dg + bnn_ref[...]


def kernel(node_feat, net_feat, pin_feat, pins_edge_index, pinned_edge_index,
           W_gc, b_gc, W_lin, b_lin, b_nn):
    idx1 = pins_edge_index.astype(jnp.int32)
    idx2 = pinned_edge_index.astype(jnp.int32)
    src, dst = idx1[0], idx1[1]
    s2, d2 = idx2[0], idx2[1]

    od_flat, src_h = _build_sc_phase1()(src, s2, net_feat)

    x = pl.pallas_call(
        _tc_x_body,
        out_shape=jax.ShapeDtypeStruct((N_NODE, H_NODE), jnp.float32),
    )(od_flat.reshape(NC, N_NODE), node_feat)

    w2 = W_lin.reshape(H_PIN * H_NET, O_NODE)
    b2 = b_lin.reshape(H_NET, O_NODE)
    eye = jnp.eye(H_PIN, dtype=jnp.float32)
    a_sel = jnp.repeat(eye, H_NET, axis=1)
    c_sel = jnp.tile(eye, (1, H_PIN))
    msg = pl.pallas_call(
        _tc_msg_body,
        grid=(E // _TBM,),
        in_specs=[
            pl.BlockSpec((_TBM, H_PIN), lambda i: (i, 0)),
            pl.BlockSpec((_TBM, H_NET), lambda i: (i, 0)),
            pl.BlockSpec((H_PIN, H_PIN * H_NET), lambda i: (0, 0)),
            pl.BlockSpec((H_NET, H_PIN * H_NET), lambda i: (0, 0)),
            pl.BlockSpec((H_PIN * H_NET, O_NODE), lambda i: (0, 0)),
            pl.BlockSpec((H_NET, O_NODE), lambda i: (0, 0)),
        ],
        out_specs=pl.BlockSpec((_TBM, O_NODE), lambda i: (i, 0)),
        out_shape=jax.ShapeDtypeStruct((E, O_NODE), jnp.float32),
    )(pin_feat, src_h, a_sel, c_sel, w2, b2)

    aggnet_f, aggnode_f, deg_flat = _build_sc_phase2()(src, dst, d2, x, msg)
    aggnet_p = aggnet_f.reshape(NC, N_NET, O_NET)
    aggnode_p = aggnode_f.reshape(NC, N_NODE, O_NODE)

    h_net, h_node = pl.pallas_call(
        _tc_final_body,
        out_shape=[
            jax.ShapeDtypeStruct((N_NET, O_NET), jnp.float32),
            jax.ShapeDtypeStruct((N_NODE, O_NODE), jnp.float32),
        ],
    )(aggnet_p, deg_flat.reshape(NC * 2, N_NET), aggnode_p,
      W_gc, b_gc.reshape(1, O_NET), b_nn.reshape(1, O_NODE))

    return (h_node, h_net)
